# Initial kernel scaffold; baseline (speedup 1.0000x reference)
import jax
import jax.numpy as jnp
from jax.experimental import pallas as pl


def kernel(pred_bar_logits, pred_pat_logits, pred_quads, orig_target_sizes):
    raise NotImplementedError("write your pallas kernel here")


# speedup vs baseline: 15.2329x; 15.2329x over previous
import functools

import jax
import jax.numpy as jnp
from jax import lax
from jax.experimental import pallas as pl
from jax.experimental.pallas import tpu as pltpu
from jax.experimental.pallas import tpu_sc as plsc

NB = 16
NCLS = 80
NQ = 20000
NFLAT = NQ * NCLS
K = 300
KPAD = 304
CHUNK = 8000
NPAIR = NFLAT // (2 * CHUNK)
CAP = 24576
BIG = 0x7FFFFFFF
F32_NEG_INF = float("-inf")
F32_POS_INF = float("inf")


def _kmap(v):
    i = plsc.bitcast(v, jnp.int32)
    s = lax.shift_right_arithmetic(i, 31)
    return jnp.bitwise_xor(i, jnp.bitwise_and(s, jnp.int32(0x7FFFFFFF)))


def _body(bar_hbm, pat_hbm, quads_hbm, scale_hbm,
          qout_hbm, blab_hbm, bsc_hbm, plab_hbm, psc_hbm,
          chunk0, chunk1, cand_val, cand_idx, hist, htot, cge,
          win_val, win_idx, out_val, out_idx, lab_buf, qidx, colbase,
          qrows, qflat, patt, smi, smf, sem0, sem1, semg):
    cc = lax.axis_index("c")
    ss = lax.axis_index("s")
    wid = ss * 2 + cc
    is_bar = wid < NB
    b = lax.rem(wid, NB)
    lane = lax.iota(jnp.int32, 16)
    zeros16 = jnp.zeros((16,), jnp.int32)

    src_base = b * NFLAT

    smi[0] = jnp.int32(0)
    smf[0] = jnp.float32(F32_NEG_INF)

    def _prefill(j, _):
        win_val[pl.ds(j * 16, 16)] = jnp.full((16,), F32_POS_INF, jnp.float32)
        out_idx[pl.ds(j * 16, 16)] = zeros16
        out_val[pl.ds(j * 16, 16)] = jnp.zeros((16,), jnp.float32)
        return 0
    lax.fori_loop(0, KPAD // 16, _prefill, 0)

    def _start_copy(buf, sem, off):
        @pl.when(is_bar)
        def _():
            pltpu.async_copy(bar_hbm.at[pl.ds(src_base + off, CHUNK)], buf, sem)
        @pl.when(jnp.logical_not(is_bar))
        def _():
            pltpu.async_copy(pat_hbm.at[pl.ds(src_base + off, CHUNK)], buf, sem)

    def _wait_copy(buf, sem):
        pltpu.make_async_copy(bar_hbm.at[pl.ds(0, CHUNK)], buf, sem).wait()

    def _process_chunk(buf, base_idx):
        tau = smf[0]
        off0 = smi[0]

        def vstep(j, offv):
            v = buf[pl.ds(j * 16, 16)]
            m = v > tau
            cum = plsc.cumsum(jnp.where(m, jnp.int32(1), jnp.int32(0)))
            pos = offv + cum - 1
            plsc.store_scatter(cand_val, [pos], v, mask=m)
            iv = lane + (base_idx + j * 16)
            plsc.store_scatter(cand_idx, [pos], iv, mask=m)
            return offv + plsc.all_reduce_population_count(m)

        offv = lax.fori_loop(0, CHUNK // 16, vstep,
                             jnp.full((16,), off0, jnp.int32))
        smi[0] = jnp.max(offv)

    def _refine():
        n = smi[0]
        nv = lax.div(n + 15, jnp.int32(16))

        pref = jnp.int32(0)
        kk = jnp.int32(K)
        for lvl in range(4):
            sh = 24 - 8 * lvl

            def hclr(i, _):
                hist[pl.ds(i * 16, 16)] = zeros16
                return 0
            lax.fori_loop(0, 256, hclr, 0)

            def hstep(j, _):
                v = cand_val[pl.ds(j * 16, 16)]
                key = _kmap(v)
                valid = (j * 16 + lane) < n
                if lvl == 0:
                    elig = valid
                    d = lax.shift_right_arithmetic(key, 24) + 128
                else:
                    elig = valid & (lax.shift_right_arithmetic(key, sh + 8) == pref)
                    d = jnp.bitwise_and(lax.shift_right_arithmetic(key, sh),
                                        jnp.int32(0xFF)) if sh else jnp.bitwise_and(key, jnp.int32(0xFF))
                hidx = lane * 256 + d
                plsc.addupdate_scatter(hist, [hidx],
                                       jnp.ones((16,), jnp.int32), mask=elig)
                return 0
            lax.fori_loop(0, nv, hstep, 0)

            def coll(i, _):
                def inner(l, a):
                    return a + hist[pl.ds(l * 256 + i * 16, 16)]
                htot[pl.ds(i * 16, 16)] = lax.fori_loop(0, 16, inner, zeros16)
                return 0
            lax.fori_loop(0, 16, coll, 0)

            def sfx(t, carry):
                i = 15 - t
                x = htot[pl.ds(i * 16, 16)]
                ssum = lax.rev(plsc.cumsum(lax.rev(x, (0,))), (0,)) + carry
                cge[pl.ds(i * 16, 16)] = ssum
                return carry + jnp.sum(x)
            lax.fori_loop(0, 16, sfx, jnp.int32(0))

            def fnd(i, best):
                cg = cge[pl.ds(i * 16, 16)]
                dd = lane + i * 16
                return jnp.maximum(best, jnp.max(jnp.where(cg >= kk, dd, -1)))
            dstar = lax.fori_loop(0, 16, fnd, jnp.int32(-1))

            def rdat(i, acc):
                cg = cge[pl.ds(i * 16, 16)]
                ht = htot[pl.ds(i * 16, 16)]
                dd = lane + i * 16
                hit = dd == dstar
                return (jnp.maximum(acc[0], jnp.max(jnp.where(hit, cg, 0))),
                        jnp.maximum(acc[1], jnp.max(jnp.where(hit, ht, 0))))
            cge_at, htot_at = lax.fori_loop(0, 16, rdat,
                                            (jnp.int32(0), jnp.int32(0)))
            kk = kk - (cge_at - htot_at)
            if lvl == 0:
                pref = dstar - 128
            else:
                pref = pref * 256 + dstar
        k300 = pref

        def cstep(j, st):
            off, eqc = st
            v = cand_val[pl.ds(j * 16, 16)]
            ii = cand_idx[pl.ds(j * 16, 16)]
            key = _kmap(v)
            valid = (j * 16 + lane) < n
            m_gt = valid & (key > k300)
            m_eq = valid & (key == k300)
            eqrank = eqc + plsc.cumsum(jnp.where(m_eq, jnp.int32(1), jnp.int32(0)))
            m = m_gt | (m_eq & (eqrank <= kk))
            pos = off + plsc.cumsum(jnp.where(m, jnp.int32(1), jnp.int32(0))) - 1
            plsc.store_scatter(win_val, [pos], v, mask=m)
            plsc.store_scatter(win_idx, [pos], ii, mask=m)
            return (off + plsc.all_reduce_population_count(m),
                    eqc + plsc.all_reduce_population_count(m_eq))
        lax.fori_loop(0, nv, cstep, (zeros16, zeros16))

        def cb(j, _):
            cand_val[pl.ds(j * 16, 16)] = win_val[pl.ds(j * 16, 16)]
            cand_idx[pl.ds(j * 16, 16)] = win_idx[pl.ds(j * 16, 16)]
            return 0
        lax.fori_loop(0, KPAD // 16, cb, 0)
        smi[0] = jnp.int32(K)

        def mn(j, a):
            return jnp.minimum(a, win_val[pl.ds(j * 16, 16)])
        tau = jnp.min(lax.fori_loop(0, KPAD // 16, mn,
                                    jnp.full((16,), F32_POS_INF, jnp.float32)))
        smf[0] = tau

    _start_copy(chunk0, sem0, 0)

    def pair(p, _):
        c0 = 2 * p * CHUNK
        _wait_copy(chunk0, sem0)
        _start_copy(chunk1, sem1, c0 + CHUNK)
        _process_chunk(chunk0, c0)
        _wait_copy(chunk1, sem1)

        @pl.when(p + 1 < NPAIR)
        def _():
            _start_copy(chunk0, sem0, c0 + 2 * CHUNK)
        _process_chunk(chunk1, c0 + CHUNK)

        @pl.when(smi[0] > CAP - 2 * CHUNK)
        def _():
            _refine()
        return 0
    lax.fori_loop(0, NPAIR, pair, 0)

    _refine()

    v = win_val[pl.ds(288, 16)]
    win_val[pl.ds(288, 16)] = jnp.where(lane + 288 >= K,
                                        jnp.float32(F32_NEG_INF), v)

    def step(t, _):
        def mx(j, a):
            return jnp.maximum(a, win_val[pl.ds(j * 16, 16)])
        m_val = jnp.max(lax.fori_loop(0, KPAD // 16, mx,
                                      jnp.full((16,), F32_NEG_INF, jnp.float32)))

        def mi(j, a):
            wv = win_val[pl.ds(j * 16, 16)]
            wi = win_idx[pl.ds(j * 16, 16)]
            return jnp.minimum(a, jnp.where(wv == m_val, wi, BIG))
        m_idx = jnp.min(lax.fori_loop(0, KPAD // 16, mi,
                                      jnp.full((16,), BIG, jnp.int32)))

        tsplat = jnp.full((16,), t, jnp.int32)
        l0 = lane == 0
        plsc.store_scatter(out_val, [tsplat],
                           jnp.full((16,), m_val, jnp.float32), mask=l0)
        plsc.store_scatter(out_idx, [tsplat],
                           jnp.full((16,), m_idx, jnp.int32), mask=l0)

        def kl(j, _):
            wv = win_val[pl.ds(j * 16, 16)]
            wi = win_idx[pl.ds(j * 16, 16)]
            hit = (wv == m_val) & (wi == m_idx)
            win_val[pl.ds(j * 16, 16)] = jnp.where(
                hit, jnp.float32(F32_NEG_INF), wv)
            return 0
        lax.fori_loop(0, KPAD // 16, kl, 0)
        return 0
    lax.fori_loop(0, K, step, 0)

    def post(j, _):
        vv = out_val[pl.ds(j * 16, 16)]
        out_val[pl.ds(j * 16, 16)] = 1.0 / (1.0 + jnp.exp(-vv))
        ii = out_idx[pl.ds(j * 16, 16)]
        lab_buf[pl.ds(j * 16, 16)] = lax.rem(ii, jnp.int32(NCLS))
        q = lax.div(ii, jnp.int32(NCLS))
        q = jnp.clip(q, 0, NQ - 1)
        g = b * NQ + q
        qidx[pl.ds(j * 16, 16)] = lax.shift_right_arithmetic(g, 4)
        colbase[pl.ds(j * 16, 16)] = jnp.bitwise_and(q, jnp.int32(15)) * 8
        return 0
    lax.fori_loop(0, KPAD // 16, post, 0)

    @pl.when(is_bar)
    def _():
        cp0 = pltpu.async_copy(quads_hbm.at[qidx.at[pl.ds(0, 128)]],
                               qrows.at[pl.ds(0, 128)], semg)
        cp1 = pltpu.async_copy(quads_hbm.at[qidx.at[pl.ds(128, 128)]],
                               qrows.at[pl.ds(128, 128)], semg)
        cp2 = pltpu.async_copy(quads_hbm.at[qidx.at[pl.ds(256, 48)]],
                               qrows.at[pl.ds(256, 48)], semg)
        pltpu.sync_copy(scale_hbm.at[pl.ds(b * 16, 16)], patt)
        cp0.wait()
        cp1.wait()
        cp2.wait()
        pat16 = patt[...]

        def qstep(t, _):
            p = t * 16 + lane
            wi = lax.shift_right_arithmetic(p, 3)
            f = jnp.bitwise_and(p, jnp.int32(7))
            col = plsc.load_gather(colbase, [wi]) + f
            g = plsc.load_gather(qrows, [wi, col])
            qflat[pl.ds(t * 16, 16)] = g * pat16
            return 0
        lax.fori_loop(0, KPAD * 8 // 16, qstep, 0)

        pltpu.sync_copy(qflat, qout_hbm.at[pl.ds(b * KPAD * 8, KPAD * 8)])
        pltpu.sync_copy(out_val, bsc_hbm.at[pl.ds(b * KPAD, KPAD)])
        pltpu.sync_copy(lab_buf, blab_hbm.at[pl.ds(b * KPAD, KPAD)])

    @pl.when(jnp.logical_not(is_bar))
    def _():
        pltpu.sync_copy(out_val, psc_hbm.at[pl.ds(b * KPAD, KPAD)])
        pltpu.sync_copy(lab_buf, plab_hbm.at[pl.ds(b * KPAD, KPAD)])


@jax.jit
def _run(bar1d, pat1d, quads2, scale16):
    f32, i32 = jnp.float32, jnp.int32
    mesh = plsc.VectorSubcoreMesh(core_axis_name="c", subcore_axis_name="s")
    return pl.kernel(
        _body,
        out_type=[
            jax.ShapeDtypeStruct((NB * KPAD * 8,), f32),
            jax.ShapeDtypeStruct((NB * KPAD,), i32),
            jax.ShapeDtypeStruct((NB * KPAD,), f32),
            jax.ShapeDtypeStruct((NB * KPAD,), i32),
            jax.ShapeDtypeStruct((NB * KPAD,), f32),
        ],
        mesh=mesh,
        compiler_params=pltpu.CompilerParams(needs_layout_passes=False),
        scratch_types=[
            pltpu.VMEM((CHUNK,), f32),
            pltpu.VMEM((CHUNK,), f32),
            pltpu.VMEM((CAP,), f32),
            pltpu.VMEM((CAP,), i32),
            pltpu.VMEM((4096,), i32),
            pltpu.VMEM((256,), i32),
            pltpu.VMEM((256,), i32),
            pltpu.VMEM((KPAD,), f32),
            pltpu.VMEM((KPAD,), i32),
            pltpu.VMEM((KPAD,), f32),
            pltpu.VMEM((KPAD,), i32),
            pltpu.VMEM((KPAD,), i32),
            pltpu.VMEM((KPAD,), i32),
            pltpu.VMEM((KPAD,), i32),
            pltpu.VMEM((KPAD, 128), f32),
            pltpu.VMEM((KPAD * 8,), f32),
            pltpu.VMEM((16,), f32),
            pltpu.SMEM((8,), i32),
            pltpu.SMEM((8,), f32),
            pltpu.SemaphoreType.DMA,
            pltpu.SemaphoreType.DMA,
            pltpu.SemaphoreType.DMA,
        ],
        name="rtdetr_post_topk_sc",
    )(bar1d, pat1d, quads2, scale16)


def kernel(pred_bar_logits, pred_pat_logits, pred_quads, orig_target_sizes):
    bar1d = pred_bar_logits.reshape(-1)
    pat1d = pred_pat_logits.reshape(-1)
    quads2 = pred_quads.reshape(NB * NQ * 8 // 128, 128)
    scale16 = jnp.tile(orig_target_sizes, (1, 8)).reshape(-1)
    qout, blab, bsc, plab, psc = _run(bar1d, pat1d, quads2, scale16)
    quads = qout.reshape(NB, KPAD, 8)[:, :K, :]
    return (quads,
            blab.reshape(NB, KPAD)[:, :K],
            bsc.reshape(NB, KPAD)[:, :K],
            plab.reshape(NB, KPAD)[:, :K],
            psc.reshape(NB, KPAD)[:, :K])


# speedup vs baseline: 26.3010x; 1.7266x over previous
import functools

import jax
import jax.numpy as jnp
from jax import lax
from jax.experimental import pallas as pl
from jax.experimental.pallas import tpu as pltpu
from jax.experimental.pallas import tpu_sc as plsc

NB = 16
NCLS = 80
NQ = 20000
NFLAT = NQ * NCLS
K = 300
KPAD = 304
CHUNK = 8000
NPAIR = NFLAT // (2 * CHUNK)
CAP = 24576
BIG = 0x7FFFFFFF
F32_NEG_INF = float("-inf")
F32_POS_INF = float("inf")


def _kmap(v):
    i = plsc.bitcast(v, jnp.int32)
    s = lax.shift_right_arithmetic(i, 31)
    return jnp.bitwise_xor(i, jnp.bitwise_and(s, jnp.int32(0x7FFFFFFF)))


def _body(bar_hbm, pat_hbm, quads_hbm, scale_hbm,
          qout_hbm, blab_hbm, bsc_hbm, plab_hbm, psc_hbm,
          chunk0, chunk1, cand_val, cand_idx, hist, htot, cge,
          win_val, win_idx, out_val, out_idx, lab_buf, qidx, colbase,
          qrows, qflat, patt, smi, smf, sem0, sem1, semg):
    cc = lax.axis_index("c")
    ss = lax.axis_index("s")
    wid = ss * 2 + cc
    is_bar = wid < NB
    b = lax.rem(wid, NB)
    lane = lax.iota(jnp.int32, 16)
    zeros16 = jnp.zeros((16,), jnp.int32)

    src_base = b * NFLAT

    smi[0] = jnp.int32(0)
    smf[0] = jnp.float32(F32_NEG_INF)

    def _prefill(j, _):
        win_val[pl.ds(j * 16, 16)] = jnp.full((16,), F32_POS_INF, jnp.float32)
        out_idx[pl.ds(j * 16, 16)] = zeros16
        out_val[pl.ds(j * 16, 16)] = jnp.zeros((16,), jnp.float32)
        return 0
    lax.fori_loop(0, KPAD // 16, _prefill, 0)

    def _start_copy(buf, sem, off):
        @pl.when(is_bar)
        def _():
            pltpu.async_copy(bar_hbm.at[pl.ds(src_base + off, CHUNK)], buf, sem)
        @pl.when(jnp.logical_not(is_bar))
        def _():
            pltpu.async_copy(pat_hbm.at[pl.ds(src_base + off, CHUNK)], buf, sem)

    def _wait_copy(buf, sem):
        pltpu.make_async_copy(bar_hbm.at[pl.ds(0, CHUNK)], buf, sem).wait()

    UNROLL = 10
    ones16 = jnp.ones((16,), jnp.int32)

    def _process_chunk(buf, base_idx):
        tau = smf[0]
        off0 = smi[0]

        def vstep(g, offm1):
            vs = [buf[pl.ds((g * UNROLL + k) * 16, 16)] for k in range(UNROLL)]
            for k in range(UNROLL):
                v = vs[k]
                m = v > tau
                pos = offm1 + plsc.cumsum(ones16, mask=m)
                plsc.store_scatter(cand_val, [pos], v, mask=m)
                iv = lane + (base_idx + (g * UNROLL + k) * 16)
                plsc.store_scatter(cand_idx, [pos], iv, mask=m)
                offm1 = offm1 + plsc.all_reduce_population_count(m)
            return offm1

        offm1 = lax.fori_loop(0, CHUNK // (16 * UNROLL), vstep,
                              jnp.full((16,), off0 - 1, jnp.int32))
        smi[0] = jnp.max(offm1) + 1

    def _refine():
        n = smi[0]
        nv = lax.div(n + 15, jnp.int32(16))

        pref = jnp.int32(0)
        kk = jnp.int32(K)
        for lvl in range(4):
            sh = 24 - 8 * lvl

            def hclr(i, _):
                hist[pl.ds(i * 16, 16)] = zeros16
                return 0
            lax.fori_loop(0, 256, hclr, 0)

            def hstep(j, _):
                v = cand_val[pl.ds(j * 16, 16)]
                key = _kmap(v)
                valid = (j * 16 + lane) < n
                if lvl == 0:
                    elig = valid
                    d = lax.shift_right_arithmetic(key, 24) + 128
                else:
                    elig = valid & (lax.shift_right_arithmetic(key, sh + 8) == pref)
                    d = jnp.bitwise_and(lax.shift_right_arithmetic(key, sh),
                                        jnp.int32(0xFF)) if sh else jnp.bitwise_and(key, jnp.int32(0xFF))
                hidx = lane * 256 + d
                plsc.addupdate_scatter(hist, [hidx],
                                       jnp.ones((16,), jnp.int32), mask=elig)
                return 0
            lax.fori_loop(0, nv, hstep, 0)

            def coll(i, _):
                def inner(l, a):
                    return a + hist[pl.ds(l * 256 + i * 16, 16)]
                htot[pl.ds(i * 16, 16)] = lax.fori_loop(0, 16, inner, zeros16)
                return 0
            lax.fori_loop(0, 16, coll, 0)

            def sfx(t, carry):
                i = 15 - t
                x = htot[pl.ds(i * 16, 16)]
                ssum = lax.rev(plsc.cumsum(lax.rev(x, (0,))), (0,)) + carry
                cge[pl.ds(i * 16, 16)] = ssum
                return carry + jnp.sum(x)
            lax.fori_loop(0, 16, sfx, jnp.int32(0))

            def fnd(i, best):
                cg = cge[pl.ds(i * 16, 16)]
                dd = lane + i * 16
                return jnp.maximum(best, jnp.max(jnp.where(cg >= kk, dd, -1)))
            dstar = lax.fori_loop(0, 16, fnd, jnp.int32(-1))

            def rdat(i, acc):
                cg = cge[pl.ds(i * 16, 16)]
                ht = htot[pl.ds(i * 16, 16)]
                dd = lane + i * 16
                hit = dd == dstar
                return (jnp.maximum(acc[0], jnp.max(jnp.where(hit, cg, 0))),
                        jnp.maximum(acc[1], jnp.max(jnp.where(hit, ht, 0))))
            cge_at, htot_at = lax.fori_loop(0, 16, rdat,
                                            (jnp.int32(0), jnp.int32(0)))
            kk = kk - (cge_at - htot_at)
            if lvl == 0:
                pref = dstar - 128
            else:
                pref = pref * 256 + dstar
        k300 = pref

        def cstep(j, st):
            off, eqc = st
            v = cand_val[pl.ds(j * 16, 16)]
            ii = cand_idx[pl.ds(j * 16, 16)]
            key = _kmap(v)
            valid = (j * 16 + lane) < n
            m_gt = valid & (key > k300)
            m_eq = valid & (key == k300)
            eqrank = eqc + plsc.cumsum(jnp.where(m_eq, jnp.int32(1), jnp.int32(0)))
            m = m_gt | (m_eq & (eqrank <= kk))
            pos = off + plsc.cumsum(jnp.where(m, jnp.int32(1), jnp.int32(0))) - 1
            plsc.store_scatter(win_val, [pos], v, mask=m)
            plsc.store_scatter(win_idx, [pos], ii, mask=m)
            return (off + plsc.all_reduce_population_count(m),
                    eqc + plsc.all_reduce_population_count(m_eq))
        lax.fori_loop(0, nv, cstep, (zeros16, zeros16))

        def cb(j, _):
            cand_val[pl.ds(j * 16, 16)] = win_val[pl.ds(j * 16, 16)]
            cand_idx[pl.ds(j * 16, 16)] = win_idx[pl.ds(j * 16, 16)]
            return 0
        lax.fori_loop(0, KPAD // 16, cb, 0)
        smi[0] = jnp.int32(K)

        def mn(j, a):
            return jnp.minimum(a, win_val[pl.ds(j * 16, 16)])
        tau = jnp.min(lax.fori_loop(0, KPAD // 16, mn,
                                    jnp.full((16,), F32_POS_INF, jnp.float32)))
        smf[0] = tau

    _start_copy(chunk0, sem0, 0)

    def pair(p, _):
        c0 = 2 * p * CHUNK
        _wait_copy(chunk0, sem0)
        _start_copy(chunk1, sem1, c0 + CHUNK)
        _process_chunk(chunk0, c0)
        _wait_copy(chunk1, sem1)

        @pl.when(p + 1 < NPAIR)
        def _():
            _start_copy(chunk0, sem0, c0 + 2 * CHUNK)
        _process_chunk(chunk1, c0 + CHUNK)

        @pl.when(smi[0] > CAP - 2 * CHUNK)
        def _():
            _refine()
        return 0
    lax.fori_loop(0, NPAIR, pair, 0)

    _refine()

    v = win_val[pl.ds(288, 16)]
    win_val[pl.ds(288, 16)] = jnp.where(lane + 288 >= K,
                                        jnp.float32(F32_NEG_INF), v)

    def step(t, _):
        wvs = [win_val[pl.ds(j * 16, 16)] for j in range(KPAD // 16)]
        acc = wvs[0]
        for j in range(1, KPAD // 16):
            acc = jnp.maximum(acc, wvs[j])
        m_val = jnp.max(acc)

        wis = [win_idx[pl.ds(j * 16, 16)] for j in range(KPAD // 16)]
        acc2 = jnp.where(wvs[0] == m_val, wis[0], BIG)
        for j in range(1, KPAD // 16):
            acc2 = jnp.minimum(acc2, jnp.where(wvs[j] == m_val, wis[j], BIG))
        m_idx = jnp.min(acc2)

        tsplat = jnp.full((16,), t, jnp.int32)
        l0 = lane == 0
        plsc.store_scatter(out_val, [tsplat],
                           jnp.full((16,), m_val, jnp.float32), mask=l0)
        plsc.store_scatter(out_idx, [tsplat],
                           jnp.full((16,), m_idx, jnp.int32), mask=l0)

        for j in range(KPAD // 16):
            hit = (wvs[j] == m_val) & (wis[j] == m_idx)
            win_val[pl.ds(j * 16, 16)] = jnp.where(
                hit, jnp.float32(F32_NEG_INF), wvs[j])
        return 0
    lax.fori_loop(0, K, step, 0)

    def post(j, _):
        vv = out_val[pl.ds(j * 16, 16)]
        out_val[pl.ds(j * 16, 16)] = 1.0 / (1.0 + jnp.exp(-vv))
        ii = out_idx[pl.ds(j * 16, 16)]
        lab_buf[pl.ds(j * 16, 16)] = lax.rem(ii, jnp.int32(NCLS))
        q = lax.div(ii, jnp.int32(NCLS))
        q = jnp.clip(q, 0, NQ - 1)
        g = b * NQ + q
        qidx[pl.ds(j * 16, 16)] = lax.shift_right_arithmetic(g, 4)
        colbase[pl.ds(j * 16, 16)] = jnp.bitwise_and(q, jnp.int32(15)) * 8
        return 0
    lax.fori_loop(0, KPAD // 16, post, 0)

    @pl.when(is_bar)
    def _():
        cp0 = pltpu.async_copy(quads_hbm.at[qidx.at[pl.ds(0, 128)]],
                               qrows.at[pl.ds(0, 128)], semg)
        cp1 = pltpu.async_copy(quads_hbm.at[qidx.at[pl.ds(128, 128)]],
                               qrows.at[pl.ds(128, 128)], semg)
        cp2 = pltpu.async_copy(quads_hbm.at[qidx.at[pl.ds(256, 48)]],
                               qrows.at[pl.ds(256, 48)], semg)
        pltpu.sync_copy(scale_hbm.at[pl.ds(b * 16, 16)], patt)
        cp0.wait()
        cp1.wait()
        cp2.wait()
        pat16 = patt[...]

        def qstep(t, _):
            p = t * 16 + lane
            wi = lax.shift_right_arithmetic(p, 3)
            f = jnp.bitwise_and(p, jnp.int32(7))
            col = plsc.load_gather(colbase, [wi]) + f
            g = plsc.load_gather(qrows, [wi, col])
            qflat[pl.ds(t * 16, 16)] = g * pat16
            return 0
        lax.fori_loop(0, KPAD * 8 // 16, qstep, 0)

        pltpu.sync_copy(qflat, qout_hbm.at[pl.ds(b * KPAD * 8, KPAD * 8)])
        pltpu.sync_copy(out_val, bsc_hbm.at[pl.ds(b * KPAD, KPAD)])
        pltpu.sync_copy(lab_buf, blab_hbm.at[pl.ds(b * KPAD, KPAD)])

    @pl.when(jnp.logical_not(is_bar))
    def _():
        pltpu.sync_copy(out_val, psc_hbm.at[pl.ds(b * KPAD, KPAD)])
        pltpu.sync_copy(lab_buf, plab_hbm.at[pl.ds(b * KPAD, KPAD)])


@jax.jit
def _run(bar1d, pat1d, quads2, scale16):
    f32, i32 = jnp.float32, jnp.int32
    mesh = plsc.VectorSubcoreMesh(core_axis_name="c", subcore_axis_name="s")
    return pl.kernel(
        _body,
        out_type=[
            jax.ShapeDtypeStruct((NB * KPAD * 8,), f32),
            jax.ShapeDtypeStruct((NB * KPAD,), i32),
            jax.ShapeDtypeStruct((NB * KPAD,), f32),
            jax.ShapeDtypeStruct((NB * KPAD,), i32),
            jax.ShapeDtypeStruct((NB * KPAD,), f32),
        ],
        mesh=mesh,
        compiler_params=pltpu.CompilerParams(needs_layout_passes=False),
        scratch_types=[
            pltpu.VMEM((CHUNK,), f32),
            pltpu.VMEM((CHUNK,), f32),
            pltpu.VMEM((CAP,), f32),
            pltpu.VMEM((CAP,), i32),
            pltpu.VMEM((4096,), i32),
            pltpu.VMEM((256,), i32),
            pltpu.VMEM((256,), i32),
            pltpu.VMEM((KPAD,), f32),
            pltpu.VMEM((KPAD,), i32),
            pltpu.VMEM((KPAD,), f32),
            pltpu.VMEM((KPAD,), i32),
            pltpu.VMEM((KPAD,), i32),
            pltpu.VMEM((KPAD,), i32),
            pltpu.VMEM((KPAD,), i32),
            pltpu.VMEM((KPAD, 128), f32),
            pltpu.VMEM((KPAD * 8,), f32),
            pltpu.VMEM((16,), f32),
            pltpu.SMEM((8,), i32),
            pltpu.SMEM((8,), f32),
            pltpu.SemaphoreType.DMA,
            pltpu.SemaphoreType.DMA,
            pltpu.SemaphoreType.DMA,
        ],
        name="rtdetr_post_topk_sc",
    )(bar1d, pat1d, quads2, scale16)


def kernel(pred_bar_logits, pred_pat_logits, pred_quads, orig_target_sizes):
    bar1d = pred_bar_logits.reshape(-1)
    pat1d = pred_pat_logits.reshape(-1)
    quads2 = pred_quads.reshape(NB * NQ * 8 // 128, 128)
    scale16 = jnp.tile(orig_target_sizes, (1, 8)).reshape(-1)
    qout, blab, bsc, plab, psc = _run(bar1d, pat1d, quads2, scale16)
    quads = qout.reshape(NB, KPAD, 8)[:, :K, :]
    return (quads,
            blab.reshape(NB, KPAD)[:, :K],
            bsc.reshape(NB, KPAD)[:, :K],
            plab.reshape(NB, KPAD)[:, :K],
            psc.reshape(NB, KPAD)[:, :K])


# speedup vs baseline: 47.3100x; 1.7988x over previous
import functools

import jax
import jax.numpy as jnp
from jax import lax
from jax.experimental import pallas as pl
from jax.experimental.pallas import tpu as pltpu
from jax.experimental.pallas import tpu_sc as plsc

NB = 16
NCLS = 80
NQ = 20000
NFLAT = NQ * NCLS
K = 300
KPAD = 304
RCH = 80
CHUNK = RCH * NCLS
NPAIR = NQ // (2 * RCH)
CAP = 24576
BIG = 0x7FFFFFFF
F32_NEG_INF = float("-inf")
F32_POS_INF = float("inf")


def _kmap(v):
    i = plsc.bitcast(v, jnp.int32)
    s = lax.shift_right_arithmetic(i, 31)
    return jnp.bitwise_xor(i, jnp.bitwise_and(s, jnp.int32(0x7FFFFFFF)))


def _body(bar_hbm, pat_hbm, quads_hbm, scale_hbm,
          qout_hbm, blab_hbm, bsc_hbm, plab_hbm, psc_hbm,
          chunk0, chunk1, cand_val, cand_idx, hist, htot, cge,
          win_val, win_idx, out_val, out_idx, lab_buf, qidx, colbase,
          qrows, qflat, patt, smi, smf, sem0, sem1, semg):
    cc = lax.axis_index("c")
    ss = lax.axis_index("s")
    wid = ss * 2 + cc
    is_bar = wid < NB
    b = lax.rem(wid, NB)
    lane = lax.iota(jnp.int32, 16)
    zeros16 = jnp.zeros((16,), jnp.int32)


    smi[0] = jnp.int32(0)
    smf[0] = jnp.float32(F32_NEG_INF)

    def _prefill(j, _):
        win_val[pl.ds(j * 16, 16)] = jnp.full((16,), F32_POS_INF, jnp.float32)
        out_idx[pl.ds(j * 16, 16)] = zeros16
        out_val[pl.ds(j * 16, 16)] = jnp.zeros((16,), jnp.float32)
        return 0
    lax.fori_loop(0, KPAD // 16, _prefill, 0)

    def _start_copy(buf, sem, row0):
        @pl.when(is_bar)
        def _():
            pltpu.async_copy(bar_hbm.at[b, pl.ds(row0, RCH)], buf, sem)
        @pl.when(jnp.logical_not(is_bar))
        def _():
            pltpu.async_copy(pat_hbm.at[b, pl.ds(row0, RCH)], buf, sem)

    def _wait_copy(buf, sem):
        pltpu.make_async_copy(bar_hbm.at[0, pl.ds(0, RCH)], buf, sem).wait()

    ones16 = jnp.ones((16,), jnp.int32)

    def _process_chunk(buf, base_row):
        tau = smf[0]
        off0 = smi[0]

        def vstep(g, offm1):
            vs = [(r, k, buf[2 * g + r, pl.ds(k * 16, 16)])
                  for r in range(2) for k in range(5)]
            for r, k, v in vs:
                m = v > tau
                pos = offm1 + plsc.cumsum(ones16, mask=m)
                plsc.store_scatter(cand_val, [pos], v, mask=m)
                iv = lane + ((base_row + 2 * g + r) * NCLS + k * 16)
                plsc.store_scatter(cand_idx, [pos], iv, mask=m)
                offm1 = offm1 + plsc.all_reduce_population_count(m)
            return offm1

        offm1 = lax.fori_loop(0, RCH // 2, vstep,
                              jnp.full((16,), off0 - 1, jnp.int32))
        smi[0] = jnp.max(offm1) + 1

    def _refine():
        n = smi[0]
        nv = lax.div(n + 15, jnp.int32(16))

        pref = jnp.int32(0)
        kk = jnp.int32(K)
        for lvl in range(4):
            sh = 24 - 8 * lvl

            def hclr(i, _):
                hist[pl.ds(i * 16, 16)] = zeros16
                return 0
            lax.fori_loop(0, 256, hclr, 0)

            def hstep(j, _):
                v = cand_val[pl.ds(j * 16, 16)]
                key = _kmap(v)
                valid = (j * 16 + lane) < n
                if lvl == 0:
                    elig = valid
                    d = lax.shift_right_arithmetic(key, 24) + 128
                else:
                    elig = valid & (lax.shift_right_arithmetic(key, sh + 8) == pref)
                    d = jnp.bitwise_and(lax.shift_right_arithmetic(key, sh),
                                        jnp.int32(0xFF)) if sh else jnp.bitwise_and(key, jnp.int32(0xFF))
                hidx = lane * 256 + d
                plsc.addupdate_scatter(hist, [hidx],
                                       jnp.ones((16,), jnp.int32), mask=elig)
                return 0
            lax.fori_loop(0, nv, hstep, 0)

            def coll(i, _):
                def inner(l, a):
                    return a + hist[pl.ds(l * 256 + i * 16, 16)]
                htot[pl.ds(i * 16, 16)] = lax.fori_loop(0, 16, inner, zeros16)
                return 0
            lax.fori_loop(0, 16, coll, 0)

            def sfx(t, carry):
                i = 15 - t
                x = htot[pl.ds(i * 16, 16)]
                ssum = lax.rev(plsc.cumsum(lax.rev(x, (0,))), (0,)) + carry
                cge[pl.ds(i * 16, 16)] = ssum
                return carry + jnp.sum(x)
            lax.fori_loop(0, 16, sfx, jnp.int32(0))

            def fnd(i, best):
                cg = cge[pl.ds(i * 16, 16)]
                dd = lane + i * 16
                return jnp.maximum(best, jnp.max(jnp.where(cg >= kk, dd, -1)))
            dstar = lax.fori_loop(0, 16, fnd, jnp.int32(-1))

            def rdat(i, acc):
                cg = cge[pl.ds(i * 16, 16)]
                ht = htot[pl.ds(i * 16, 16)]
                dd = lane + i * 16
                hit = dd == dstar
                return (jnp.maximum(acc[0], jnp.max(jnp.where(hit, cg, 0))),
                        jnp.maximum(acc[1], jnp.max(jnp.where(hit, ht, 0))))
            cge_at, htot_at = lax.fori_loop(0, 16, rdat,
                                            (jnp.int32(0), jnp.int32(0)))
            kk = kk - (cge_at - htot_at)
            if lvl == 0:
                pref = dstar - 128
            else:
                pref = pref * 256 + dstar
        k300 = pref

        def cstep(j, st):
            off, eqc = st
            v = cand_val[pl.ds(j * 16, 16)]
            ii = cand_idx[pl.ds(j * 16, 16)]
            key = _kmap(v)
            valid = (j * 16 + lane) < n
            m_gt = valid & (key > k300)
            m_eq = valid & (key == k300)
            eqrank = eqc + plsc.cumsum(jnp.where(m_eq, jnp.int32(1), jnp.int32(0)))
            m = m_gt | (m_eq & (eqrank <= kk))
            pos = off + plsc.cumsum(jnp.where(m, jnp.int32(1), jnp.int32(0))) - 1
            plsc.store_scatter(win_val, [pos], v, mask=m)
            plsc.store_scatter(win_idx, [pos], ii, mask=m)
            return (off + plsc.all_reduce_population_count(m),
                    eqc + plsc.all_reduce_population_count(m_eq))
        lax.fori_loop(0, nv, cstep, (zeros16, zeros16))

        def cb(j, _):
            cand_val[pl.ds(j * 16, 16)] = win_val[pl.ds(j * 16, 16)]
            cand_idx[pl.ds(j * 16, 16)] = win_idx[pl.ds(j * 16, 16)]
            return 0
        lax.fori_loop(0, KPAD // 16, cb, 0)
        smi[0] = jnp.int32(K)

        def mn(j, a):
            return jnp.minimum(a, win_val[pl.ds(j * 16, 16)])
        tau = jnp.min(lax.fori_loop(0, KPAD // 16, mn,
                                    jnp.full((16,), F32_POS_INF, jnp.float32)))
        smf[0] = tau

    _start_copy(chunk0, sem0, 0)

    def pair(p, _):
        r0 = 2 * p * RCH
        _wait_copy(chunk0, sem0)
        _start_copy(chunk1, sem1, r0 + RCH)
        _process_chunk(chunk0, r0)
        _wait_copy(chunk1, sem1)

        @pl.when(p + 1 < NPAIR)
        def _():
            _start_copy(chunk0, sem0, r0 + 2 * RCH)
        _process_chunk(chunk1, r0 + RCH)

        @pl.when(smi[0] > CAP - 2 * CHUNK)
        def _():
            _refine()
        return 0
    lax.fori_loop(0, NPAIR, pair, 0)

    _refine()

    v = win_val[pl.ds(288, 16)]
    win_val[pl.ds(288, 16)] = jnp.where(lane + 288 >= K,
                                        jnp.float32(F32_NEG_INF), v)

    def step(t, _):
        wvs = [win_val[pl.ds(j * 16, 16)] for j in range(KPAD // 16)]
        acc = wvs[0]
        for j in range(1, KPAD // 16):
            acc = jnp.maximum(acc, wvs[j])
        m_val = jnp.max(acc)

        wis = [win_idx[pl.ds(j * 16, 16)] for j in range(KPAD // 16)]
        acc2 = jnp.where(wvs[0] == m_val, wis[0], BIG)
        for j in range(1, KPAD // 16):
            acc2 = jnp.minimum(acc2, jnp.where(wvs[j] == m_val, wis[j], BIG))
        m_idx = jnp.min(acc2)

        tsplat = jnp.full((16,), t, jnp.int32)
        l0 = lane == 0
        plsc.store_scatter(out_val, [tsplat],
                           jnp.full((16,), m_val, jnp.float32), mask=l0)
        plsc.store_scatter(out_idx, [tsplat],
                           jnp.full((16,), m_idx, jnp.int32), mask=l0)

        for j in range(KPAD // 16):
            hit = (wvs[j] == m_val) & (wis[j] == m_idx)
            win_val[pl.ds(j * 16, 16)] = jnp.where(
                hit, jnp.float32(F32_NEG_INF), wvs[j])
        return 0
    lax.fori_loop(0, K, step, 0)

    def post(j, _):
        vv = out_val[pl.ds(j * 16, 16)]
        out_val[pl.ds(j * 16, 16)] = 1.0 / (1.0 + jnp.exp(-vv))
        ii = out_idx[pl.ds(j * 16, 16)]
        lab_buf[pl.ds(j * 16, 16)] = lax.rem(ii, jnp.int32(NCLS))
        q = lax.div(ii, jnp.int32(NCLS))
        q = jnp.clip(q, 0, NQ - 1)
        g = b * NQ + q
        qidx[pl.ds(j * 16, 16)] = lax.shift_right_arithmetic(g, 4)
        colbase[pl.ds(j * 16, 16)] = jnp.bitwise_and(q, jnp.int32(15)) * 8
        return 0
    lax.fori_loop(0, KPAD // 16, post, 0)

    @pl.when(is_bar)
    def _():
        cp0 = pltpu.async_copy(quads_hbm.at[qidx.at[pl.ds(0, 128)]],
                               qrows.at[pl.ds(0, 128)], semg)
        cp1 = pltpu.async_copy(quads_hbm.at[qidx.at[pl.ds(128, 128)]],
                               qrows.at[pl.ds(128, 128)], semg)
        cp2 = pltpu.async_copy(quads_hbm.at[qidx.at[pl.ds(256, 48)]],
                               qrows.at[pl.ds(256, 48)], semg)
        pltpu.sync_copy(scale_hbm.at[pl.ds(b * 16, 16)], patt)
        cp0.wait()
        cp1.wait()
        cp2.wait()
        pat16 = patt[...]

        def qstep(t, _):
            p = t * 16 + lane
            wi = lax.shift_right_arithmetic(p, 3)
            f = jnp.bitwise_and(p, jnp.int32(7))
            col = plsc.load_gather(colbase, [wi]) + f
            g = plsc.load_gather(qrows, [wi, col])
            qflat[pl.ds(t * 16, 16)] = g * pat16
            return 0
        lax.fori_loop(0, KPAD * 8 // 16, qstep, 0)

        pltpu.sync_copy(qflat, qout_hbm.at[pl.ds(b * KPAD * 8, KPAD * 8)])
        pltpu.sync_copy(out_val, bsc_hbm.at[pl.ds(b * KPAD, KPAD)])
        pltpu.sync_copy(lab_buf, blab_hbm.at[pl.ds(b * KPAD, KPAD)])

    @pl.when(jnp.logical_not(is_bar))
    def _():
        pltpu.sync_copy(out_val, psc_hbm.at[pl.ds(b * KPAD, KPAD)])
        pltpu.sync_copy(lab_buf, plab_hbm.at[pl.ds(b * KPAD, KPAD)])


@jax.jit
def _run(bar1d, pat1d, quads2, scale16):
    f32, i32 = jnp.float32, jnp.int32
    mesh = plsc.VectorSubcoreMesh(core_axis_name="c", subcore_axis_name="s")
    return pl.kernel(
        _body,
        out_type=[
            jax.ShapeDtypeStruct((NB * KPAD * 8,), f32),
            jax.ShapeDtypeStruct((NB * KPAD,), i32),
            jax.ShapeDtypeStruct((NB * KPAD,), f32),
            jax.ShapeDtypeStruct((NB * KPAD,), i32),
            jax.ShapeDtypeStruct((NB * KPAD,), f32),
        ],
        mesh=mesh,
        compiler_params=pltpu.CompilerParams(needs_layout_passes=False),
        scratch_types=[
            pltpu.VMEM((RCH, NCLS), f32),
            pltpu.VMEM((RCH, NCLS), f32),
            pltpu.VMEM((CAP,), f32),
            pltpu.VMEM((CAP,), i32),
            pltpu.VMEM((4096,), i32),
            pltpu.VMEM((256,), i32),
            pltpu.VMEM((256,), i32),
            pltpu.VMEM((KPAD,), f32),
            pltpu.VMEM((KPAD,), i32),
            pltpu.VMEM((KPAD,), f32),
            pltpu.VMEM((KPAD,), i32),
            pltpu.VMEM((KPAD,), i32),
            pltpu.VMEM((KPAD,), i32),
            pltpu.VMEM((KPAD,), i32),
            pltpu.VMEM((KPAD, 128), f32),
            pltpu.VMEM((KPAD * 8,), f32),
            pltpu.VMEM((16,), f32),
            pltpu.SMEM((8,), i32),
            pltpu.SMEM((8,), f32),
            pltpu.SemaphoreType.DMA,
            pltpu.SemaphoreType.DMA,
            pltpu.SemaphoreType.DMA,
        ],
        name="rtdetr_post_topk_sc",
    )(bar1d, pat1d, quads2, scale16)


def kernel(pred_bar_logits, pred_pat_logits, pred_quads, orig_target_sizes):
    quads2 = pred_quads.reshape(NB * NQ * 8 // 128, 128)
    scale16 = jnp.tile(orig_target_sizes, (1, 8)).reshape(-1)
    qout, blab, bsc, plab, psc = _run(
        pred_bar_logits, pred_pat_logits, quads2, scale16)
    quads = qout.reshape(NB, KPAD, 8)[:, :K, :]
    return (quads,
            blab.reshape(NB, KPAD)[:, :K],
            bsc.reshape(NB, KPAD)[:, :K],
            plab.reshape(NB, KPAD)[:, :K],
            psc.reshape(NB, KPAD)[:, :K])


# speedup vs baseline: 49.2302x; 1.0406x over previous
import functools

import jax
import jax.numpy as jnp
from jax import lax
from jax.experimental import pallas as pl
from jax.experimental.pallas import tpu as pltpu
from jax.experimental.pallas import tpu_sc as plsc

NB = 16
NCLS = 80
NQ = 20000
NFLAT = NQ * NCLS
K = 300
KPAD = 304
RCH = 80
CHUNK = RCH * NCLS
NPAIR = NQ // (2 * RCH)
CAP = 24576
BIG = 0x7FFFFFFF
F32_NEG_INF = float("-inf")
F32_POS_INF = float("inf")


def _kmap(v):
    i = plsc.bitcast(v, jnp.int32)
    s = lax.shift_right_arithmetic(i, 31)
    return jnp.bitwise_xor(i, jnp.bitwise_and(s, jnp.int32(0x7FFFFFFF)))


def _body(bar_hbm, pat_hbm, quads_hbm, scale_hbm,
          qout_hbm, blab_hbm, bsc_hbm, plab_hbm, psc_hbm,
          chunk0, chunk1, cand_val, cand_idx, hist, htot, cge,
          win_val, win_idx, out_val, out_idx, lab_buf, qidx,
          qrows, qflat, patt, smi, smf, sem0, sem1, semg):
    cc = lax.axis_index("c")
    ss = lax.axis_index("s")
    wid = ss * 2 + cc
    is_bar = wid < NB
    b = lax.rem(wid, NB)
    lane = lax.iota(jnp.int32, 16)
    zeros16 = jnp.zeros((16,), jnp.int32)


    smi[0] = jnp.int32(0)
    smf[0] = jnp.float32(F32_NEG_INF)

    def _prefill(j, _):
        win_val[pl.ds(j * 16, 16)] = jnp.full((16,), F32_POS_INF, jnp.float32)
        out_idx[pl.ds(j * 16, 16)] = zeros16
        out_val[pl.ds(j * 16, 16)] = jnp.zeros((16,), jnp.float32)
        return 0
    lax.fori_loop(0, KPAD // 16, _prefill, 0)

    def _start_copy(buf, sem, row0):
        @pl.when(is_bar)
        def _():
            pltpu.async_copy(bar_hbm.at[b, pl.ds(row0, RCH)], buf, sem)
        @pl.when(jnp.logical_not(is_bar))
        def _():
            pltpu.async_copy(pat_hbm.at[b, pl.ds(row0, RCH)], buf, sem)

    def _wait_copy(buf, sem):
        pltpu.make_async_copy(bar_hbm.at[0, pl.ds(0, RCH)], buf, sem).wait()

    ones16 = jnp.ones((16,), jnp.int32)

    def _process_chunk(buf, base_row):
        tau = smf[0]
        off0 = smi[0]

        def vstep(g, offm1):
            vs = [(r, k, buf[2 * g + r, pl.ds(k * 16, 16)])
                  for r in range(2) for k in range(5)]
            for r, k, v in vs:
                m = v > tau
                pos = offm1 + plsc.cumsum(ones16, mask=m)
                plsc.store_scatter(cand_val, [pos], v, mask=m)
                iv = lane + ((base_row + 2 * g + r) * NCLS + k * 16)
                plsc.store_scatter(cand_idx, [pos], iv, mask=m)
                offm1 = offm1 + plsc.all_reduce_population_count(m)
            return offm1

        offm1 = lax.fori_loop(0, RCH // 2, vstep,
                              jnp.full((16,), off0 - 1, jnp.int32))
        smi[0] = jnp.max(offm1) + 1

    def _refine():
        n = smi[0]
        nv = lax.div(n + 15, jnp.int32(16))

        pref = jnp.int32(0)
        kk = jnp.int32(K)
        for lvl in range(4):
            sh = 24 - 8 * lvl

            def hclr(i, _):
                hist[pl.ds(i * 16, 16)] = zeros16
                return 0
            lax.fori_loop(0, 256, hclr, 0)

            def hstep(j, _):
                v = cand_val[pl.ds(j * 16, 16)]
                key = _kmap(v)
                valid = (j * 16 + lane) < n
                if lvl == 0:
                    elig = valid
                    d = lax.shift_right_arithmetic(key, 24) + 128
                else:
                    elig = valid & (lax.shift_right_arithmetic(key, sh + 8) == pref)
                    d = jnp.bitwise_and(lax.shift_right_arithmetic(key, sh),
                                        jnp.int32(0xFF)) if sh else jnp.bitwise_and(key, jnp.int32(0xFF))
                hidx = lane * 256 + d
                plsc.addupdate_scatter(hist, [hidx],
                                       jnp.ones((16,), jnp.int32), mask=elig)
                return 0
            lax.fori_loop(0, nv, hstep, 0)

            def coll(i, _):
                def inner(l, a):
                    return a + hist[pl.ds(l * 256 + i * 16, 16)]
                htot[pl.ds(i * 16, 16)] = lax.fori_loop(0, 16, inner, zeros16)
                return 0
            lax.fori_loop(0, 16, coll, 0)

            def sfx(t, carry):
                i = 15 - t
                x = htot[pl.ds(i * 16, 16)]
                ssum = lax.rev(plsc.cumsum(lax.rev(x, (0,))), (0,)) + carry
                cge[pl.ds(i * 16, 16)] = ssum
                return carry + jnp.sum(x)
            lax.fori_loop(0, 16, sfx, jnp.int32(0))

            def fnd(i, best):
                cg = cge[pl.ds(i * 16, 16)]
                dd = lane + i * 16
                return jnp.maximum(best, jnp.max(jnp.where(cg >= kk, dd, -1)))
            dstar = lax.fori_loop(0, 16, fnd, jnp.int32(-1))

            def rdat(i, acc):
                cg = cge[pl.ds(i * 16, 16)]
                ht = htot[pl.ds(i * 16, 16)]
                dd = lane + i * 16
                hit = dd == dstar
                return (jnp.maximum(acc[0], jnp.max(jnp.where(hit, cg, 0))),
                        jnp.maximum(acc[1], jnp.max(jnp.where(hit, ht, 0))))
            cge_at, htot_at = lax.fori_loop(0, 16, rdat,
                                            (jnp.int32(0), jnp.int32(0)))
            kk = kk - (cge_at - htot_at)
            if lvl == 0:
                pref = dstar - 128
            else:
                pref = pref * 256 + dstar
        k300 = pref

        def cstep(j, st):
            off, eqc = st
            v = cand_val[pl.ds(j * 16, 16)]
            ii = cand_idx[pl.ds(j * 16, 16)]
            key = _kmap(v)
            valid = (j * 16 + lane) < n
            m_gt = valid & (key > k300)
            m_eq = valid & (key == k300)
            eqrank = eqc + plsc.cumsum(jnp.where(m_eq, jnp.int32(1), jnp.int32(0)))
            m = m_gt | (m_eq & (eqrank <= kk))
            pos = off + plsc.cumsum(jnp.where(m, jnp.int32(1), jnp.int32(0))) - 1
            plsc.store_scatter(win_val, [pos], v, mask=m)
            plsc.store_scatter(win_idx, [pos], ii, mask=m)
            return (off + plsc.all_reduce_population_count(m),
                    eqc + plsc.all_reduce_population_count(m_eq))
        lax.fori_loop(0, nv, cstep, (zeros16, zeros16))

        def cb(j, _):
            cand_val[pl.ds(j * 16, 16)] = win_val[pl.ds(j * 16, 16)]
            cand_idx[pl.ds(j * 16, 16)] = win_idx[pl.ds(j * 16, 16)]
            return 0
        lax.fori_loop(0, KPAD // 16, cb, 0)
        smi[0] = jnp.int32(K)

        def mn(j, a):
            return jnp.minimum(a, win_val[pl.ds(j * 16, 16)])
        tau = jnp.min(lax.fori_loop(0, KPAD // 16, mn,
                                    jnp.full((16,), F32_POS_INF, jnp.float32)))
        smf[0] = tau

    _start_copy(chunk0, sem0, 0)

    def pair(p, _):
        r0 = 2 * p * RCH
        _wait_copy(chunk0, sem0)
        _start_copy(chunk1, sem1, r0 + RCH)
        _process_chunk(chunk0, r0)
        _wait_copy(chunk1, sem1)

        @pl.when(p + 1 < NPAIR)
        def _():
            _start_copy(chunk0, sem0, r0 + 2 * RCH)
        _process_chunk(chunk1, r0 + RCH)

        @pl.when(smi[0] > CAP - 2 * CHUNK)
        def _():
            _refine()
        return 0
    lax.fori_loop(0, NPAIR, pair, 0)

    _refine()

    v = win_val[pl.ds(288, 16)]
    win_val[pl.ds(288, 16)] = jnp.where(lane + 288 >= K,
                                        jnp.float32(F32_NEG_INF), v)

    def step(t, _):
        wvs = [win_val[pl.ds(j * 16, 16)] for j in range(KPAD // 16)]
        acc = wvs[0]
        for j in range(1, KPAD // 16):
            acc = jnp.maximum(acc, wvs[j])
        m_val = jnp.max(acc)

        wis = [win_idx[pl.ds(j * 16, 16)] for j in range(KPAD // 16)]
        acc2 = jnp.where(wvs[0] == m_val, wis[0], BIG)
        for j in range(1, KPAD // 16):
            acc2 = jnp.minimum(acc2, jnp.where(wvs[j] == m_val, wis[j], BIG))
        m_idx = jnp.min(acc2)

        tsplat = jnp.full((16,), t, jnp.int32)
        l0 = lane == 0
        plsc.store_scatter(out_val, [tsplat],
                           jnp.full((16,), m_val, jnp.float32), mask=l0)
        plsc.store_scatter(out_idx, [tsplat],
                           jnp.full((16,), m_idx, jnp.int32), mask=l0)

        for j in range(KPAD // 16):
            hit = (wvs[j] == m_val) & (wis[j] == m_idx)
            win_val[pl.ds(j * 16, 16)] = jnp.where(
                hit, jnp.float32(F32_NEG_INF), wvs[j])
        return 0
    lax.fori_loop(0, K, step, 0)

    def post(j, _):
        vv = out_val[pl.ds(j * 16, 16)]
        out_val[pl.ds(j * 16, 16)] = 1.0 / (1.0 + jnp.exp(-vv))
        ii = out_idx[pl.ds(j * 16, 16)]
        lab_buf[pl.ds(j * 16, 16)] = lax.rem(ii, jnp.int32(NCLS))
        q = lax.div(ii, jnp.int32(NCLS))
        qidx[pl.ds(j * 16, 16)] = jnp.clip(q, 0, NQ - 1)
        return 0
    lax.fori_loop(0, KPAD // 16, post, 0)

    @pl.when(is_bar)
    def _():
        def fire(t, _):
            qv = qidx[pl.ds(jnp.bitwise_and(t, -16), 16)]
            qt = jnp.max(jnp.where(lane == jnp.bitwise_and(t, 15), qv, 0))
            pltpu.async_copy(quads_hbm.at[b, qt], qrows.at[t], semg)
            return 0
        lax.fori_loop(0, KPAD, fire, 0)
        pltpu.sync_copy(scale_hbm.at[pl.ds(b * 16, 16)], patt)
        pltpu.make_async_copy(quads_hbm.at[0, pl.ds(0, KPAD)], qrows, semg).wait()
        pat16 = patt[...]

        def qstep(t, _):
            p = t * 16 + lane
            wi = lax.shift_right_arithmetic(p, 3)
            f = jnp.bitwise_and(p, jnp.int32(7))
            g = plsc.load_gather(qrows, [wi, f])
            qflat[pl.ds(t * 16, 16)] = g * pat16
            return 0
        lax.fori_loop(0, KPAD * 8 // 16, qstep, 0)

        pltpu.sync_copy(qflat, qout_hbm.at[pl.ds(b * KPAD * 8, KPAD * 8)])
        pltpu.sync_copy(out_val, bsc_hbm.at[pl.ds(b * KPAD, KPAD)])
        pltpu.sync_copy(lab_buf, blab_hbm.at[pl.ds(b * KPAD, KPAD)])

    @pl.when(jnp.logical_not(is_bar))
    def _():
        pltpu.sync_copy(out_val, psc_hbm.at[pl.ds(b * KPAD, KPAD)])
        pltpu.sync_copy(lab_buf, plab_hbm.at[pl.ds(b * KPAD, KPAD)])


@jax.jit
def _run(bar1d, pat1d, quads2, scale16):
    f32, i32 = jnp.float32, jnp.int32
    mesh = plsc.VectorSubcoreMesh(core_axis_name="c", subcore_axis_name="s")
    return pl.kernel(
        _body,
        out_type=[
            jax.ShapeDtypeStruct((NB * KPAD * 8,), f32),
            jax.ShapeDtypeStruct((NB * KPAD,), i32),
            jax.ShapeDtypeStruct((NB * KPAD,), f32),
            jax.ShapeDtypeStruct((NB * KPAD,), i32),
            jax.ShapeDtypeStruct((NB * KPAD,), f32),
        ],
        mesh=mesh,
        compiler_params=pltpu.CompilerParams(needs_layout_passes=False),
        scratch_types=[
            pltpu.VMEM((RCH, NCLS), f32),
            pltpu.VMEM((RCH, NCLS), f32),
            pltpu.VMEM((CAP,), f32),
            pltpu.VMEM((CAP,), i32),
            pltpu.VMEM((4096,), i32),
            pltpu.VMEM((256,), i32),
            pltpu.VMEM((256,), i32),
            pltpu.VMEM((KPAD,), f32),
            pltpu.VMEM((KPAD,), i32),
            pltpu.VMEM((KPAD,), f32),
            pltpu.VMEM((KPAD,), i32),
            pltpu.VMEM((KPAD,), i32),
            pltpu.VMEM((KPAD,), i32),
            pltpu.VMEM((KPAD, 8), f32),
            pltpu.VMEM((KPAD * 8,), f32),
            pltpu.VMEM((16,), f32),
            pltpu.SMEM((8,), i32),
            pltpu.SMEM((8,), f32),
            pltpu.SemaphoreType.DMA,
            pltpu.SemaphoreType.DMA,
            pltpu.SemaphoreType.DMA,
        ],
        name="rtdetr_post_topk_sc",
    )(bar1d, pat1d, quads2, scale16)


def kernel(pred_bar_logits, pred_pat_logits, pred_quads, orig_target_sizes):
    scale16 = jnp.tile(orig_target_sizes, (1, 8)).reshape(-1)
    qout, blab, bsc, plab, psc = _run(
        pred_bar_logits, pred_pat_logits, pred_quads, scale16)
    quads = qout.reshape(NB, KPAD, 8)[:, :K, :]
    return (quads,
            blab.reshape(NB, KPAD)[:, :K],
            bsc.reshape(NB, KPAD)[:, :K],
            plab.reshape(NB, KPAD)[:, :K],
            psc.reshape(NB, KPAD)[:, :K])


# speedup vs baseline: 52.6467x; 1.0694x over previous
import functools

import jax
import jax.numpy as jnp
from jax import lax
from jax.experimental import pallas as pl
from jax.experimental.pallas import tpu as pltpu
from jax.experimental.pallas import tpu_sc as plsc

NB = 16
NCLS = 80
NQ = 20000
NFLAT = NQ * NCLS
K = 300
KPAD = 304
RCH = 80
CHUNK = RCH * NCLS
NBUF = 5
NGRP = NQ // (NBUF * RCH)
CAP = 12288
BIG = 0x7FFFFFFF
F32_NEG_INF = float("-inf")
F32_POS_INF = float("inf")


def _kmap(v):
    i = plsc.bitcast(v, jnp.int32)
    s = lax.shift_right_arithmetic(i, 31)
    return jnp.bitwise_xor(i, jnp.bitwise_and(s, jnp.int32(0x7FFFFFFF)))


def _body(bar_hbm, pat_hbm, quads_hbm, scale_hbm,
          qout_hbm, blab_hbm, bsc_hbm, plab_hbm, psc_hbm,
          chunk0, chunk1, chunk2, chunk3, chunk4,
          cand_val, cand_idx, hist, htot, cge,
          win_val, win_idx, out_val, out_idx, lab_buf, qidx,
          qrows, qflat, patt, smi, smf,
          sem0, sem1, sem2, sem3, sem4, semg):
    bufs = (chunk0, chunk1, chunk2, chunk3, chunk4)
    sems = (sem0, sem1, sem2, sem3, sem4)
    cc = lax.axis_index("c")
    ss = lax.axis_index("s")
    wid = ss * 2 + cc
    is_bar = wid < NB
    b = lax.rem(wid, NB)
    lane = lax.iota(jnp.int32, 16)
    zeros16 = jnp.zeros((16,), jnp.int32)


    smi[0] = jnp.int32(0)
    smf[0] = jnp.float32(F32_NEG_INF)

    def _prefill(j, _):
        win_val[pl.ds(j * 16, 16)] = jnp.full((16,), F32_POS_INF, jnp.float32)
        out_idx[pl.ds(j * 16, 16)] = zeros16
        out_val[pl.ds(j * 16, 16)] = jnp.zeros((16,), jnp.float32)
        return 0
    lax.fori_loop(0, KPAD // 16, _prefill, 0)

    def _start_copy(buf, sem, row0):
        @pl.when(is_bar)
        def _():
            pltpu.async_copy(bar_hbm.at[b, pl.ds(row0, RCH)], buf, sem)
        @pl.when(jnp.logical_not(is_bar))
        def _():
            pltpu.async_copy(pat_hbm.at[b, pl.ds(row0, RCH)], buf, sem)

    def _wait_copy(buf, sem):
        pltpu.make_async_copy(bar_hbm.at[0, pl.ds(0, RCH)], buf, sem).wait()

    ones16 = jnp.ones((16,), jnp.int32)

    def _process_chunk(buf, base_row):
        tau = smf[0]
        off0 = smi[0]

        def vstep(g, offm1):
            vs = [(r, k, buf[2 * g + r, pl.ds(k * 16, 16)])
                  for r in range(2) for k in range(5)]
            for r, k, v in vs:
                m = v > tau
                pos = offm1 + plsc.cumsum(ones16, mask=m)
                plsc.store_scatter(cand_val, [pos], v, mask=m)
                iv = lane + ((base_row + 2 * g + r) * NCLS + k * 16)
                plsc.store_scatter(cand_idx, [pos], iv, mask=m)
                offm1 = offm1 + plsc.all_reduce_population_count(m)
            return offm1

        offm1 = lax.fori_loop(0, RCH // 2, vstep,
                              jnp.full((16,), off0 - 1, jnp.int32))
        smi[0] = jnp.max(offm1) + 1

    def _refine():
        n = smi[0]
        nv = lax.div(n + 15, jnp.int32(16))

        pref = jnp.int32(0)
        kk = jnp.int32(K)
        for lvl in range(4):
            sh = 24 - 8 * lvl

            def hclr(i, _):
                hist[pl.ds(i * 16, 16)] = zeros16
                return 0
            lax.fori_loop(0, 256, hclr, 0)

            def hstep(j, _):
                v = cand_val[pl.ds(j * 16, 16)]
                key = _kmap(v)
                valid = (j * 16 + lane) < n
                if lvl == 0:
                    elig = valid
                    d = lax.shift_right_arithmetic(key, 24) + 128
                else:
                    elig = valid & (lax.shift_right_arithmetic(key, sh + 8) == pref)
                    d = jnp.bitwise_and(lax.shift_right_arithmetic(key, sh),
                                        jnp.int32(0xFF)) if sh else jnp.bitwise_and(key, jnp.int32(0xFF))
                hidx = lane * 256 + d
                plsc.addupdate_scatter(hist, [hidx],
                                       jnp.ones((16,), jnp.int32), mask=elig)
                return 0
            lax.fori_loop(0, nv, hstep, 0)

            def coll(i, _):
                def inner(l, a):
                    return a + hist[pl.ds(l * 256 + i * 16, 16)]
                htot[pl.ds(i * 16, 16)] = lax.fori_loop(0, 16, inner, zeros16)
                return 0
            lax.fori_loop(0, 16, coll, 0)

            def sfx(t, carry):
                i = 15 - t
                x = htot[pl.ds(i * 16, 16)]
                ssum = lax.rev(plsc.cumsum(lax.rev(x, (0,))), (0,)) + carry
                cge[pl.ds(i * 16, 16)] = ssum
                return carry + jnp.sum(x)
            lax.fori_loop(0, 16, sfx, jnp.int32(0))

            def fnd(i, best):
                cg = cge[pl.ds(i * 16, 16)]
                dd = lane + i * 16
                return jnp.maximum(best, jnp.max(jnp.where(cg >= kk, dd, -1)))
            dstar = lax.fori_loop(0, 16, fnd, jnp.int32(-1))

            def rdat(i, acc):
                cg = cge[pl.ds(i * 16, 16)]
                ht = htot[pl.ds(i * 16, 16)]
                dd = lane + i * 16
                hit = dd == dstar
                return (jnp.maximum(acc[0], jnp.max(jnp.where(hit, cg, 0))),
                        jnp.maximum(acc[1], jnp.max(jnp.where(hit, ht, 0))))
            cge_at, htot_at = lax.fori_loop(0, 16, rdat,
                                            (jnp.int32(0), jnp.int32(0)))
            kk = kk - (cge_at - htot_at)
            if lvl == 0:
                pref = dstar - 128
            else:
                pref = pref * 256 + dstar
        k300 = pref

        def cstep(j, st):
            off, eqc = st
            v = cand_val[pl.ds(j * 16, 16)]
            ii = cand_idx[pl.ds(j * 16, 16)]
            key = _kmap(v)
            valid = (j * 16 + lane) < n
            m_gt = valid & (key > k300)
            m_eq = valid & (key == k300)
            eqrank = eqc + plsc.cumsum(jnp.where(m_eq, jnp.int32(1), jnp.int32(0)))
            m = m_gt | (m_eq & (eqrank <= kk))
            pos = off + plsc.cumsum(jnp.where(m, jnp.int32(1), jnp.int32(0))) - 1
            plsc.store_scatter(win_val, [pos], v, mask=m)
            plsc.store_scatter(win_idx, [pos], ii, mask=m)
            return (off + plsc.all_reduce_population_count(m),
                    eqc + plsc.all_reduce_population_count(m_eq))
        lax.fori_loop(0, nv, cstep, (zeros16, zeros16))

        def cb(j, _):
            cand_val[pl.ds(j * 16, 16)] = win_val[pl.ds(j * 16, 16)]
            cand_idx[pl.ds(j * 16, 16)] = win_idx[pl.ds(j * 16, 16)]
            return 0
        lax.fori_loop(0, KPAD // 16, cb, 0)
        smi[0] = jnp.int32(K)

        def mn(j, a):
            return jnp.minimum(a, win_val[pl.ds(j * 16, 16)])
        tau = jnp.min(lax.fori_loop(0, KPAD // 16, mn,
                                    jnp.full((16,), F32_POS_INF, jnp.float32)))
        smf[0] = tau

    for i in range(NBUF):
        _start_copy(bufs[i], sems[i], i * RCH)

    def group(gg, _):
        r0 = gg * NBUF * RCH
        for i in range(NBUF):
            _wait_copy(bufs[i], sems[i])
            _process_chunk(bufs[i], r0 + i * RCH)

            @pl.when(r0 + (i + NBUF) * RCH < NQ)
            def _():
                _start_copy(bufs[i], sems[i], r0 + (i + NBUF) * RCH)

            @pl.when(smi[0] > CAP - CHUNK)
            def _():
                _refine()
        return 0
    lax.fori_loop(0, NGRP, group, 0)

    _refine()

    v = win_val[pl.ds(288, 16)]
    win_val[pl.ds(288, 16)] = jnp.where(lane + 288 >= K,
                                        jnp.float32(F32_NEG_INF), v)

    def step(t, _):
        wvs = [win_val[pl.ds(j * 16, 16)] for j in range(KPAD // 16)]
        acc = wvs[0]
        for j in range(1, KPAD // 16):
            acc = jnp.maximum(acc, wvs[j])
        m_val = jnp.max(acc)

        wis = [win_idx[pl.ds(j * 16, 16)] for j in range(KPAD // 16)]
        acc2 = jnp.where(wvs[0] == m_val, wis[0], BIG)
        for j in range(1, KPAD // 16):
            acc2 = jnp.minimum(acc2, jnp.where(wvs[j] == m_val, wis[j], BIG))
        m_idx = jnp.min(acc2)

        tsplat = jnp.full((16,), t, jnp.int32)
        l0 = lane == 0
        plsc.store_scatter(out_val, [tsplat],
                           jnp.full((16,), m_val, jnp.float32), mask=l0)
        plsc.store_scatter(out_idx, [tsplat],
                           jnp.full((16,), m_idx, jnp.int32), mask=l0)

        for j in range(KPAD // 16):
            hit = (wvs[j] == m_val) & (wis[j] == m_idx)
            win_val[pl.ds(j * 16, 16)] = jnp.where(
                hit, jnp.float32(F32_NEG_INF), wvs[j])
        return 0
    lax.fori_loop(0, K, step, 0)

    def post(j, _):
        vv = out_val[pl.ds(j * 16, 16)]
        out_val[pl.ds(j * 16, 16)] = 1.0 / (1.0 + jnp.exp(-vv))
        ii = out_idx[pl.ds(j * 16, 16)]
        lab_buf[pl.ds(j * 16, 16)] = lax.rem(ii, jnp.int32(NCLS))
        q = lax.div(ii, jnp.int32(NCLS))
        qidx[pl.ds(j * 16, 16)] = jnp.clip(q, 0, NQ - 1)
        return 0
    lax.fori_loop(0, KPAD // 16, post, 0)

    @pl.when(is_bar)
    def _():
        def fire(t, _):
            qv = qidx[pl.ds(jnp.bitwise_and(t, -16), 16)]
            qt = jnp.max(jnp.where(lane == jnp.bitwise_and(t, 15), qv, 0))
            pltpu.async_copy(quads_hbm.at[b, qt], qrows.at[t], semg)
            return 0
        lax.fori_loop(0, KPAD, fire, 0)
        pltpu.sync_copy(scale_hbm.at[pl.ds(b * 16, 16)], patt)
        pltpu.make_async_copy(quads_hbm.at[0, pl.ds(0, KPAD)], qrows, semg).wait()
        pat16 = patt[...]

        def qstep(t, _):
            p = t * 16 + lane
            wi = lax.shift_right_arithmetic(p, 3)
            f = jnp.bitwise_and(p, jnp.int32(7))
            g = plsc.load_gather(qrows, [wi, f])
            qflat[pl.ds(t * 16, 16)] = g * pat16
            return 0
        lax.fori_loop(0, KPAD * 8 // 16, qstep, 0)

        pltpu.sync_copy(qflat, qout_hbm.at[pl.ds(b * KPAD * 8, KPAD * 8)])
        pltpu.sync_copy(out_val, bsc_hbm.at[pl.ds(b * KPAD, KPAD)])
        pltpu.sync_copy(lab_buf, blab_hbm.at[pl.ds(b * KPAD, KPAD)])

    @pl.when(jnp.logical_not(is_bar))
    def _():
        pltpu.sync_copy(out_val, psc_hbm.at[pl.ds(b * KPAD, KPAD)])
        pltpu.sync_copy(lab_buf, plab_hbm.at[pl.ds(b * KPAD, KPAD)])


@jax.jit
def _run(bar1d, pat1d, quads2, scale16):
    f32, i32 = jnp.float32, jnp.int32
    mesh = plsc.VectorSubcoreMesh(core_axis_name="c", subcore_axis_name="s")
    return pl.kernel(
        _body,
        out_type=[
            jax.ShapeDtypeStruct((NB * KPAD * 8,), f32),
            jax.ShapeDtypeStruct((NB * KPAD,), i32),
            jax.ShapeDtypeStruct((NB * KPAD,), f32),
            jax.ShapeDtypeStruct((NB * KPAD,), i32),
            jax.ShapeDtypeStruct((NB * KPAD,), f32),
        ],
        mesh=mesh,
        compiler_params=pltpu.CompilerParams(needs_layout_passes=False),
        scratch_types=[
            pltpu.VMEM((RCH, NCLS), f32),
            pltpu.VMEM((RCH, NCLS), f32),
            pltpu.VMEM((RCH, NCLS), f32),
            pltpu.VMEM((RCH, NCLS), f32),
            pltpu.VMEM((RCH, NCLS), f32),
            pltpu.VMEM((CAP,), f32),
            pltpu.VMEM((CAP,), i32),
            pltpu.VMEM((4096,), i32),
            pltpu.VMEM((256,), i32),
            pltpu.VMEM((256,), i32),
            pltpu.VMEM((KPAD,), f32),
            pltpu.VMEM((KPAD,), i32),
            pltpu.VMEM((KPAD,), f32),
            pltpu.VMEM((KPAD,), i32),
            pltpu.VMEM((KPAD,), i32),
            pltpu.VMEM((KPAD,), i32),
            pltpu.VMEM((KPAD, 8), f32),
            pltpu.VMEM((KPAD * 8,), f32),
            pltpu.VMEM((16,), f32),
            pltpu.SMEM((8,), i32),
            pltpu.SMEM((8,), f32),
            pltpu.SemaphoreType.DMA,
            pltpu.SemaphoreType.DMA,
            pltpu.SemaphoreType.DMA,
            pltpu.SemaphoreType.DMA,
            pltpu.SemaphoreType.DMA,
            pltpu.SemaphoreType.DMA,
        ],
        name="rtdetr_post_topk_sc",
    )(bar1d, pat1d, quads2, scale16)


def kernel(pred_bar_logits, pred_pat_logits, pred_quads, orig_target_sizes):
    scale16 = jnp.tile(orig_target_sizes, (1, 8)).reshape(-1)
    qout, blab, bsc, plab, psc = _run(
        pred_bar_logits, pred_pat_logits, pred_quads, scale16)
    quads = qout.reshape(NB, KPAD, 8)[:, :K, :]
    return (quads,
            blab.reshape(NB, KPAD)[:, :K],
            bsc.reshape(NB, KPAD)[:, :K],
            plab.reshape(NB, KPAD)[:, :K],
            psc.reshape(NB, KPAD)[:, :K])


# speedup vs baseline: 54.7006x; 1.0390x over previous
import functools

import jax
import jax.numpy as jnp
from jax import lax
from jax.experimental import pallas as pl
from jax.experimental.pallas import tpu as pltpu
from jax.experimental.pallas import tpu_sc as plsc

NB = 16
NCLS = 80
NQ = 20000
NFLAT = NQ * NCLS
K = 300
KPAD = 304
RCH = 80
CHUNK = RCH * NCLS
NBUF = 5
NGRP = NQ // (NBUF * RCH)
CAP = 12288
BIG = 0x7FFFFFFF
F32_NEG_INF = float("-inf")
F32_POS_INF = float("inf")


def _kmap(v):
    i = plsc.bitcast(v, jnp.int32)
    s = lax.shift_right_arithmetic(i, 31)
    return jnp.bitwise_xor(i, jnp.bitwise_and(s, jnp.int32(0x7FFFFFFF)))


def _body(bar_hbm, pat_hbm, quads_hbm, scale_hbm,
          qout_hbm, blab_hbm, bsc_hbm, plab_hbm, psc_hbm,
          chunk0, chunk1, chunk2, chunk3, chunk4,
          cand_val, cand_idx, hist, htot, cge,
          win_val, win_idx, out_val, out_idx, lab_buf, qidx,
          qrows, qflat, patt, smi, smf,
          sem0, sem1, sem2, sem3, sem4, semg):
    bufs = (chunk0, chunk1, chunk2, chunk3, chunk4)
    sems = (sem0, sem1, sem2, sem3, sem4)
    cc = lax.axis_index("c")
    ss = lax.axis_index("s")
    wid = ss * 2 + cc
    is_bar = wid < NB
    b = lax.rem(wid, NB)
    lane = lax.iota(jnp.int32, 16)
    zeros16 = jnp.zeros((16,), jnp.int32)


    smi[0] = jnp.int32(0)
    smf[0] = jnp.float32(F32_NEG_INF)

    def _prefill(j, _):
        win_val[pl.ds(j * 16, 16)] = jnp.full((16,), F32_POS_INF, jnp.float32)
        out_idx[pl.ds(j * 16, 16)] = zeros16
        out_val[pl.ds(j * 16, 16)] = jnp.zeros((16,), jnp.float32)
        return 0
    lax.fori_loop(0, KPAD // 16, _prefill, 0)

    def _start_copy(buf, sem, row0):
        @pl.when(is_bar)
        def _():
            pltpu.async_copy(bar_hbm.at[b, pl.ds(row0, RCH)], buf, sem)
        @pl.when(jnp.logical_not(is_bar))
        def _():
            pltpu.async_copy(pat_hbm.at[b, pl.ds(row0, RCH)], buf, sem)

    def _wait_copy(buf, sem):
        pltpu.make_async_copy(bar_hbm.at[0, pl.ds(0, RCH)], buf, sem).wait()

    ones16 = jnp.ones((16,), jnp.int32)

    def _process_chunk(buf, base_row):
        tau = smf[0]
        off0 = smi[0]

        def vstep(g, offm1):
            vs = [(r, k, buf[4 * g + r, pl.ds(k * 16, 16)])
                  for r in range(4) for k in range(5)]
            for r, k, v in vs:
                m = v > tau
                pos = offm1 + plsc.cumsum(ones16, mask=m)
                plsc.store_scatter(cand_val, [pos], v, mask=m)
                iv = lane + ((base_row + 4 * g + r) * NCLS + k * 16)
                plsc.store_scatter(cand_idx, [pos], iv, mask=m)
                offm1 = offm1 + plsc.all_reduce_population_count(m)
            return offm1

        offm1 = lax.fori_loop(0, RCH // 4, vstep,
                              jnp.full((16,), off0 - 1, jnp.int32))
        smi[0] = jnp.max(offm1) + 1

    def _refine():
        n = smi[0]
        nv = lax.div(n + 15, jnp.int32(16))

        pref = jnp.int32(0)
        kk = jnp.int32(K)
        for lvl in range(4):
            sh = 24 - 8 * lvl

            def hclr(i, _):
                hist[pl.ds(i * 16, 16)] = zeros16
                return 0
            lax.fori_loop(0, 256, hclr, 0)

            def hstep(j, _):
                v = cand_val[pl.ds(j * 16, 16)]
                key = _kmap(v)
                valid = (j * 16 + lane) < n
                if lvl == 0:
                    elig = valid
                    d = lax.shift_right_arithmetic(key, 24) + 128
                else:
                    elig = valid & (lax.shift_right_arithmetic(key, sh + 8) == pref)
                    d = jnp.bitwise_and(lax.shift_right_arithmetic(key, sh),
                                        jnp.int32(0xFF)) if sh else jnp.bitwise_and(key, jnp.int32(0xFF))
                hidx = lane * 256 + d
                plsc.addupdate_scatter(hist, [hidx],
                                       jnp.ones((16,), jnp.int32), mask=elig)
                return 0
            lax.fori_loop(0, nv, hstep, 0)

            def coll(i, _):
                def inner(l, a):
                    return a + hist[pl.ds(l * 256 + i * 16, 16)]
                htot[pl.ds(i * 16, 16)] = lax.fori_loop(0, 16, inner, zeros16)
                return 0
            lax.fori_loop(0, 16, coll, 0)

            def sfx(t, carry):
                i = 15 - t
                x = htot[pl.ds(i * 16, 16)]
                ssum = lax.rev(plsc.cumsum(lax.rev(x, (0,))), (0,)) + carry
                cge[pl.ds(i * 16, 16)] = ssum
                return carry + jnp.sum(x)
            lax.fori_loop(0, 16, sfx, jnp.int32(0))

            def fnd(i, best):
                cg = cge[pl.ds(i * 16, 16)]
                dd = lane + i * 16
                return jnp.maximum(best, jnp.max(jnp.where(cg >= kk, dd, -1)))
            dstar = lax.fori_loop(0, 16, fnd, jnp.int32(-1))

            def rdat(i, acc):
                cg = cge[pl.ds(i * 16, 16)]
                ht = htot[pl.ds(i * 16, 16)]
                dd = lane + i * 16
                hit = dd == dstar
                return (jnp.maximum(acc[0], jnp.max(jnp.where(hit, cg, 0))),
                        jnp.maximum(acc[1], jnp.max(jnp.where(hit, ht, 0))))
            cge_at, htot_at = lax.fori_loop(0, 16, rdat,
                                            (jnp.int32(0), jnp.int32(0)))
            kk = kk - (cge_at - htot_at)
            if lvl == 0:
                pref = dstar - 128
            else:
                pref = pref * 256 + dstar
        k300 = pref

        def cstep(j, st):
            off, eqc = st
            v = cand_val[pl.ds(j * 16, 16)]
            ii = cand_idx[pl.ds(j * 16, 16)]
            key = _kmap(v)
            valid = (j * 16 + lane) < n
            m_gt = valid & (key > k300)
            m_eq = valid & (key == k300)
            eqrank = eqc + plsc.cumsum(jnp.where(m_eq, jnp.int32(1), jnp.int32(0)))
            m = m_gt | (m_eq & (eqrank <= kk))
            pos = off + plsc.cumsum(jnp.where(m, jnp.int32(1), jnp.int32(0))) - 1
            plsc.store_scatter(win_val, [pos], v, mask=m)
            plsc.store_scatter(win_idx, [pos], ii, mask=m)
            return (off + plsc.all_reduce_population_count(m),
                    eqc + plsc.all_reduce_population_count(m_eq))
        lax.fori_loop(0, nv, cstep, (zeros16, zeros16))

        def cb(j, _):
            cand_val[pl.ds(j * 16, 16)] = win_val[pl.ds(j * 16, 16)]
            cand_idx[pl.ds(j * 16, 16)] = win_idx[pl.ds(j * 16, 16)]
            return 0
        lax.fori_loop(0, KPAD // 16, cb, 0)
        smi[0] = jnp.int32(K)

        def mn(j, a):
            return jnp.minimum(a, win_val[pl.ds(j * 16, 16)])
        tau = jnp.min(lax.fori_loop(0, KPAD // 16, mn,
                                    jnp.full((16,), F32_POS_INF, jnp.float32)))
        smf[0] = tau

    for i in range(NBUF):
        _start_copy(bufs[i], sems[i], i * RCH)

    def group(gg, _):
        r0 = gg * NBUF * RCH
        for i in range(NBUF):
            _wait_copy(bufs[i], sems[i])
            _process_chunk(bufs[i], r0 + i * RCH)

            @pl.when(r0 + (i + NBUF) * RCH < NQ)
            def _():
                _start_copy(bufs[i], sems[i], r0 + (i + NBUF) * RCH)

            @pl.when(smi[0] > CAP - CHUNK)
            def _():
                _refine()
        return 0
    lax.fori_loop(0, NGRP, group, 0)

    _refine()

    v = win_val[pl.ds(288, 16)]
    win_val[pl.ds(288, 16)] = jnp.where(lane + 288 >= K,
                                        jnp.float32(F32_NEG_INF), v)

    def step(t, _):
        wvs = [win_val[pl.ds(j * 16, 16)] for j in range(KPAD // 16)]
        acc = wvs[0]
        for j in range(1, KPAD // 16):
            acc = jnp.maximum(acc, wvs[j])
        m_val = jnp.max(acc)

        wis = [win_idx[pl.ds(j * 16, 16)] for j in range(KPAD // 16)]
        acc2 = jnp.where(wvs[0] == m_val, wis[0], BIG)
        for j in range(1, KPAD // 16):
            acc2 = jnp.minimum(acc2, jnp.where(wvs[j] == m_val, wis[j], BIG))
        m_idx = jnp.min(acc2)

        tsplat = jnp.full((16,), t, jnp.int32)
        l0 = lane == 0
        plsc.store_scatter(out_val, [tsplat],
                           jnp.full((16,), m_val, jnp.float32), mask=l0)
        plsc.store_scatter(out_idx, [tsplat],
                           jnp.full((16,), m_idx, jnp.int32), mask=l0)

        for j in range(KPAD // 16):
            hit = (wvs[j] == m_val) & (wis[j] == m_idx)
            win_val[pl.ds(j * 16, 16)] = jnp.where(
                hit, jnp.float32(F32_NEG_INF), wvs[j])
        return 0
    lax.fori_loop(0, K, step, 0)

    def post(j, _):
        vv = out_val[pl.ds(j * 16, 16)]
        out_val[pl.ds(j * 16, 16)] = 1.0 / (1.0 + jnp.exp(-vv))
        ii = out_idx[pl.ds(j * 16, 16)]
        lab_buf[pl.ds(j * 16, 16)] = lax.rem(ii, jnp.int32(NCLS))
        q = lax.div(ii, jnp.int32(NCLS))
        qidx[pl.ds(j * 16, 16)] = jnp.clip(q, 0, NQ - 1)
        return 0
    lax.fori_loop(0, KPAD // 16, post, 0)

    @pl.when(is_bar)
    def _():
        def fire(t, _):
            qv = qidx[pl.ds(jnp.bitwise_and(t, -16), 16)]
            qt = jnp.max(jnp.where(lane == jnp.bitwise_and(t, 15), qv, 0))
            pltpu.async_copy(quads_hbm.at[b, qt], qrows.at[t], semg)
            return 0
        lax.fori_loop(0, KPAD, fire, 0)
        pltpu.sync_copy(scale_hbm.at[pl.ds(b * 16, 16)], patt)
        pltpu.make_async_copy(quads_hbm.at[0, pl.ds(0, KPAD)], qrows, semg).wait()
        pat16 = patt[...]

        def qstep(t, _):
            p = t * 16 + lane
            wi = lax.shift_right_arithmetic(p, 3)
            f = jnp.bitwise_and(p, jnp.int32(7))
            g = plsc.load_gather(qrows, [wi, f])
            qflat[pl.ds(t * 16, 16)] = g * pat16
            return 0
        lax.fori_loop(0, KPAD * 8 // 16, qstep, 0)

        pltpu.sync_copy(qflat, qout_hbm.at[pl.ds(b * KPAD * 8, KPAD * 8)])
        pltpu.sync_copy(out_val, bsc_hbm.at[pl.ds(b * KPAD, KPAD)])
        pltpu.sync_copy(lab_buf, blab_hbm.at[pl.ds(b * KPAD, KPAD)])

    @pl.when(jnp.logical_not(is_bar))
    def _():
        pltpu.sync_copy(out_val, psc_hbm.at[pl.ds(b * KPAD, KPAD)])
        pltpu.sync_copy(lab_buf, plab_hbm.at[pl.ds(b * KPAD, KPAD)])


@jax.jit
def _run(bar1d, pat1d, quads2, scale16):
    f32, i32 = jnp.float32, jnp.int32
    mesh = plsc.VectorSubcoreMesh(core_axis_name="c", subcore_axis_name="s")
    return pl.kernel(
        _body,
        out_type=[
            jax.ShapeDtypeStruct((NB * KPAD * 8,), f32),
            jax.ShapeDtypeStruct((NB * KPAD,), i32),
            jax.ShapeDtypeStruct((NB * KPAD,), f32),
            jax.ShapeDtypeStruct((NB * KPAD,), i32),
            jax.ShapeDtypeStruct((NB * KPAD,), f32),
        ],
        mesh=mesh,
        compiler_params=pltpu.CompilerParams(needs_layout_passes=False),
        scratch_types=[
            pltpu.VMEM((RCH, NCLS), f32),
            pltpu.VMEM((RCH, NCLS), f32),
            pltpu.VMEM((RCH, NCLS), f32),
            pltpu.VMEM((RCH, NCLS), f32),
            pltpu.VMEM((RCH, NCLS), f32),
            pltpu.VMEM((CAP,), f32),
            pltpu.VMEM((CAP,), i32),
            pltpu.VMEM((4096,), i32),
            pltpu.VMEM((256,), i32),
            pltpu.VMEM((256,), i32),
            pltpu.VMEM((KPAD,), f32),
            pltpu.VMEM((KPAD,), i32),
            pltpu.VMEM((KPAD,), f32),
            pltpu.VMEM((KPAD,), i32),
            pltpu.VMEM((KPAD,), i32),
            pltpu.VMEM((KPAD,), i32),
            pltpu.VMEM((KPAD, 8), f32),
            pltpu.VMEM((KPAD * 8,), f32),
            pltpu.VMEM((16,), f32),
            pltpu.SMEM((8,), i32),
            pltpu.SMEM((8,), f32),
            pltpu.SemaphoreType.DMA,
            pltpu.SemaphoreType.DMA,
            pltpu.SemaphoreType.DMA,
            pltpu.SemaphoreType.DMA,
            pltpu.SemaphoreType.DMA,
            pltpu.SemaphoreType.DMA,
        ],
        name="rtdetr_post_topk_sc",
    )(bar1d, pat1d, quads2, scale16)


def kernel(pred_bar_logits, pred_pat_logits, pred_quads, orig_target_sizes):
    scale16 = jnp.tile(orig_target_sizes, (1, 8)).reshape(-1)
    qout, blab, bsc, plab, psc = _run(
        pred_bar_logits, pred_pat_logits, pred_quads, scale16)
    quads = qout.reshape(NB, KPAD, 8)[:, :K, :]
    return (quads,
            blab.reshape(NB, KPAD)[:, :K],
            bsc.reshape(NB, KPAD)[:, :K],
            plab.reshape(NB, KPAD)[:, :K],
            psc.reshape(NB, KPAD)[:, :K])


# speedup vs baseline: 54.7992x; 1.0018x over previous
import functools

import jax
import jax.numpy as jnp
from jax import lax
from jax.experimental import pallas as pl
from jax.experimental.pallas import tpu as pltpu
from jax.experimental.pallas import tpu_sc as plsc

NB = 16
NCLS = 80
NQ = 20000
NFLAT = NQ * NCLS
K = 300
KPAD = 304
RCH = 80
CHUNK = RCH * NCLS
NBUF = 5
NGRP = NQ // (NBUF * RCH)
CAP = 12288
BIG = 0x7FFFFFFF
F32_NEG_INF = float("-inf")
F32_POS_INF = float("inf")


def _kmap(v):
    i = plsc.bitcast(v, jnp.int32)
    s = lax.shift_right_arithmetic(i, 31)
    return jnp.bitwise_xor(i, jnp.bitwise_and(s, jnp.int32(0x7FFFFFFF)))


def _body(bar_hbm, pat_hbm, quads_hbm, scale_hbm,
          qout_hbm, blab_hbm, bsc_hbm, plab_hbm, psc_hbm,
          chunk0, chunk1, chunk2, chunk3, chunk4,
          cand_val, cand_idx, hist, htot, cge,
          win_val, win_idx, out_val, out_idx, lab_buf, qidx,
          qrows, qflat, patt, smi, smf,
          sem0, sem1, sem2, sem3, sem4, semg):
    bufs = (chunk0, chunk1, chunk2, chunk3, chunk4)
    sems = (sem0, sem1, sem2, sem3, sem4)
    cc = lax.axis_index("c")
    ss = lax.axis_index("s")
    wid = ss * 2 + cc
    is_bar = wid < NB
    b = lax.rem(wid, NB)
    lane = lax.iota(jnp.int32, 16)
    zeros16 = jnp.zeros((16,), jnp.int32)


    smi[0] = jnp.int32(0)
    smf[0] = jnp.float32(F32_NEG_INF)

    def _prefill(j, _):
        win_val[pl.ds(j * 16, 16)] = jnp.full((16,), F32_POS_INF, jnp.float32)
        out_idx[pl.ds(j * 16, 16)] = zeros16
        out_val[pl.ds(j * 16, 16)] = jnp.zeros((16,), jnp.float32)
        return 0
    lax.fori_loop(0, KPAD // 16, _prefill, 0)

    def _start_copy(buf, sem, row0):
        @pl.when(is_bar)
        def _():
            pltpu.async_copy(bar_hbm.at[b, pl.ds(row0, RCH)], buf, sem)
        @pl.when(jnp.logical_not(is_bar))
        def _():
            pltpu.async_copy(pat_hbm.at[b, pl.ds(row0, RCH)], buf, sem)

    def _wait_copy(buf, sem):
        pltpu.make_async_copy(bar_hbm.at[0, pl.ds(0, RCH)], buf, sem).wait()

    ones16 = jnp.ones((16,), jnp.int32)

    def _process_chunk(buf, base_row):
        tau = smf[0]
        off0 = smi[0]

        def vstep(g, offm1):
            vs = [(r, k, buf[4 * g + r, pl.ds(k * 16, 16)])
                  for r in range(4) for k in range(5)]
            for r, k, v in vs:
                m = v > tau
                pos = offm1 + plsc.cumsum(ones16, mask=m)
                plsc.store_scatter(cand_val, [pos], v, mask=m)
                iv = lane + ((base_row + 4 * g + r) * NCLS + k * 16)
                plsc.store_scatter(cand_idx, [pos], iv, mask=m)
                offm1 = offm1 + plsc.all_reduce_population_count(m)
            return offm1

        offm1 = lax.fori_loop(0, RCH // 4, vstep,
                              jnp.full((16,), off0 - 1, jnp.int32))
        smi[0] = jnp.max(offm1) + 1

    def _refine():
        n = smi[0]
        nv = lax.div(n + 15, jnp.int32(16))

        pref = jnp.int32(0)
        kk = jnp.int32(K)
        for lvl in range(4):
            sh = 24 - 8 * lvl

            def hclr(i, _):
                hist[pl.ds(i * 16, 16)] = zeros16
                return 0
            lax.fori_loop(0, 256, hclr, 0)

            def hstep(j, _):
                v = cand_val[pl.ds(j * 16, 16)]
                key = _kmap(v)
                valid = (j * 16 + lane) < n
                if lvl == 0:
                    elig = valid
                    d = lax.shift_right_arithmetic(key, 24) + 128
                else:
                    elig = valid & (lax.shift_right_arithmetic(key, sh + 8) == pref)
                    d = jnp.bitwise_and(lax.shift_right_arithmetic(key, sh),
                                        jnp.int32(0xFF)) if sh else jnp.bitwise_and(key, jnp.int32(0xFF))
                hidx = lane * 256 + d
                plsc.addupdate_scatter(hist, [hidx],
                                       jnp.ones((16,), jnp.int32), mask=elig)
                return 0
            lax.fori_loop(0, nv, hstep, 0)

            def coll(i, _):
                def inner(l, a):
                    return a + hist[pl.ds(l * 256 + i * 16, 16)]
                htot[pl.ds(i * 16, 16)] = lax.fori_loop(0, 16, inner, zeros16)
                return 0
            lax.fori_loop(0, 16, coll, 0)

            def sfx(t, carry):
                i = 15 - t
                x = htot[pl.ds(i * 16, 16)]
                ssum = lax.rev(plsc.cumsum(lax.rev(x, (0,))), (0,)) + carry
                cge[pl.ds(i * 16, 16)] = ssum
                return carry + jnp.sum(x)
            lax.fori_loop(0, 16, sfx, jnp.int32(0))

            def fnd(i, best):
                cg = cge[pl.ds(i * 16, 16)]
                dd = lane + i * 16
                return jnp.maximum(best, jnp.max(jnp.where(cg >= kk, dd, -1)))
            dstar = lax.fori_loop(0, 16, fnd, jnp.int32(-1))

            def rdat(i, acc):
                cg = cge[pl.ds(i * 16, 16)]
                ht = htot[pl.ds(i * 16, 16)]
                dd = lane + i * 16
                hit = dd == dstar
                return (jnp.maximum(acc[0], jnp.max(jnp.where(hit, cg, 0))),
                        jnp.maximum(acc[1], jnp.max(jnp.where(hit, ht, 0))))
            cge_at, htot_at = lax.fori_loop(0, 16, rdat,
                                            (jnp.int32(0), jnp.int32(0)))
            kk = kk - (cge_at - htot_at)
            if lvl == 0:
                pref = dstar - 128
            else:
                pref = pref * 256 + dstar
        k300 = pref

        def cstep(j, st):
            off, eqc = st
            v = cand_val[pl.ds(j * 16, 16)]
            ii = cand_idx[pl.ds(j * 16, 16)]
            key = _kmap(v)
            valid = (j * 16 + lane) < n
            m_gt = valid & (key > k300)
            m_eq = valid & (key == k300)
            eqrank = eqc + plsc.cumsum(jnp.where(m_eq, jnp.int32(1), jnp.int32(0)))
            m = m_gt | (m_eq & (eqrank <= kk))
            pos = off + plsc.cumsum(jnp.where(m, jnp.int32(1), jnp.int32(0))) - 1
            plsc.store_scatter(win_val, [pos], v, mask=m)
            plsc.store_scatter(win_idx, [pos], ii, mask=m)
            return (off + plsc.all_reduce_population_count(m),
                    eqc + plsc.all_reduce_population_count(m_eq))
        lax.fori_loop(0, nv, cstep, (zeros16, zeros16))

        def cb(j, _):
            cand_val[pl.ds(j * 16, 16)] = win_val[pl.ds(j * 16, 16)]
            cand_idx[pl.ds(j * 16, 16)] = win_idx[pl.ds(j * 16, 16)]
            return 0
        lax.fori_loop(0, KPAD // 16, cb, 0)
        smi[0] = jnp.int32(K)

        def mn(j, a):
            return jnp.minimum(a, win_val[pl.ds(j * 16, 16)])
        tau = jnp.min(lax.fori_loop(0, KPAD // 16, mn,
                                    jnp.full((16,), F32_POS_INF, jnp.float32)))
        smf[0] = tau

    for i in range(NBUF):
        _start_copy(bufs[i], sems[i], i * RCH)

    def group(gg, _):
        r0 = gg * NBUF * RCH
        for i in range(NBUF):
            _wait_copy(bufs[i], sems[i])
            _process_chunk(bufs[i], r0 + i * RCH)

            @pl.when(r0 + (i + NBUF) * RCH < NQ)
            def _():
                _start_copy(bufs[i], sems[i], r0 + (i + NBUF) * RCH)

            @pl.when(smi[0] > CAP - CHUNK)
            def _():
                _refine()
        return 0
    lax.fori_loop(0, NGRP, group, 0)

    _refine()

    v = win_val[pl.ds(288, 16)]
    win_val[pl.ds(288, 16)] = jnp.where(lane + 288 >= K,
                                        jnp.float32(F32_NEG_INF), v)

    def step(t, _):
        wvs = [win_val[pl.ds(j * 16, 16)] for j in range(KPAD // 16)]
        acc = wvs[0]
        for j in range(1, KPAD // 16):
            acc = jnp.maximum(acc, wvs[j])
        m_val = jnp.max(acc)

        wis = [win_idx[pl.ds(j * 16, 16)] for j in range(KPAD // 16)]
        acc2 = jnp.where(wvs[0] == m_val, wis[0], BIG)
        for j in range(1, KPAD // 16):
            acc2 = jnp.minimum(acc2, jnp.where(wvs[j] == m_val, wis[j], BIG))
        m_idx = jnp.min(acc2)

        tsplat = jnp.full((16,), t, jnp.int32)
        l0 = lane == 0
        plsc.store_scatter(out_val, [tsplat],
                           jnp.full((16,), m_val, jnp.float32), mask=l0)
        plsc.store_scatter(out_idx, [tsplat],
                           jnp.full((16,), m_idx, jnp.int32), mask=l0)

        for j in range(KPAD // 16):
            hit = (wvs[j] == m_val) & (wis[j] == m_idx)
            win_val[pl.ds(j * 16, 16)] = jnp.where(
                hit, jnp.float32(F32_NEG_INF), wvs[j])
        return 0
    lax.fori_loop(0, K, step, 0)

    def post(j, _):
        vv = out_val[pl.ds(j * 16, 16)]
        out_val[pl.ds(j * 16, 16)] = 1.0 / (1.0 + jnp.exp(-vv))
        ii = out_idx[pl.ds(j * 16, 16)]
        lab_buf[pl.ds(j * 16, 16)] = lax.rem(ii, jnp.int32(NCLS))
        q = lax.div(ii, jnp.int32(NCLS))
        qidx[pl.ds(j * 16, 16)] = jnp.clip(q, 0, NQ - 1)
        return 0
    lax.fori_loop(0, KPAD // 16, post, 0)

    @pl.when(is_bar)
    def _():
        def fire(t, _):
            qv = qidx[pl.ds(jnp.bitwise_and(t, -16), 16)]
            qt = jnp.max(jnp.where(lane == jnp.bitwise_and(t, 15), qv, 0))
            pltpu.async_copy(quads_hbm.at[b, qt], qrows.at[t], semg)
            return 0
        lax.fori_loop(0, KPAD, fire, 0)
        pltpu.sync_copy(scale_hbm.at[pl.ds(b * 16, 16)], patt)
        pltpu.make_async_copy(quads_hbm.at[0, pl.ds(0, KPAD)], qrows, semg).wait()
        pat16 = patt[...]

        def qstep(t, _):
            p = t * 16 + lane
            wi = lax.shift_right_arithmetic(p, 3)
            f = jnp.bitwise_and(p, jnp.int32(7))
            g = plsc.load_gather(qrows, [wi, f])
            qflat[pl.ds(t * 16, 16)] = g * pat16
            return 0
        lax.fori_loop(0, KPAD * 8 // 16, qstep, 0)

        pltpu.sync_copy(qflat, qout_hbm.at[pl.ds(b * KPAD * 8, KPAD * 8)])
        pltpu.sync_copy(out_val, bsc_hbm.at[pl.ds(b * KPAD, KPAD)])
        pltpu.sync_copy(lab_buf, blab_hbm.at[pl.ds(b * KPAD, KPAD)])

    @pl.when(jnp.logical_not(is_bar))
    def _():
        pltpu.sync_copy(out_val, psc_hbm.at[pl.ds(b * KPAD, KPAD)])
        pltpu.sync_copy(lab_buf, plab_hbm.at[pl.ds(b * KPAD, KPAD)])


@jax.jit
def _run(bar1d, pat1d, quads2, scale16):
    f32, i32 = jnp.float32, jnp.int32
    mesh = plsc.VectorSubcoreMesh(core_axis_name="c", subcore_axis_name="s")
    return pl.kernel(
        _body,
        out_type=[
            jax.ShapeDtypeStruct((NB * KPAD * 8,), f32),
            jax.ShapeDtypeStruct((NB * KPAD,), i32),
            jax.ShapeDtypeStruct((NB * KPAD,), f32),
            jax.ShapeDtypeStruct((NB * KPAD,), i32),
            jax.ShapeDtypeStruct((NB * KPAD,), f32),
        ],
        mesh=mesh,
        compiler_params=pltpu.CompilerParams(needs_layout_passes=False,
                                             use_tc_tiling_on_sc=True),
        scratch_types=[
            pltpu.VMEM((RCH, NCLS), f32),
            pltpu.VMEM((RCH, NCLS), f32),
            pltpu.VMEM((RCH, NCLS), f32),
            pltpu.VMEM((RCH, NCLS), f32),
            pltpu.VMEM((RCH, NCLS), f32),
            pltpu.VMEM((CAP,), f32),
            pltpu.VMEM((CAP,), i32),
            pltpu.VMEM((4096,), i32),
            pltpu.VMEM((256,), i32),
            pltpu.VMEM((256,), i32),
            pltpu.VMEM((KPAD,), f32),
            pltpu.VMEM((KPAD,), i32),
            pltpu.VMEM((KPAD,), f32),
            pltpu.VMEM((KPAD,), i32),
            pltpu.VMEM((KPAD,), i32),
            pltpu.VMEM((KPAD,), i32),
            pltpu.VMEM((KPAD, 8), f32),
            pltpu.VMEM((KPAD * 8,), f32),
            pltpu.VMEM((16,), f32),
            pltpu.SMEM((8,), i32),
            pltpu.SMEM((8,), f32),
            pltpu.SemaphoreType.DMA,
            pltpu.SemaphoreType.DMA,
            pltpu.SemaphoreType.DMA,
            pltpu.SemaphoreType.DMA,
            pltpu.SemaphoreType.DMA,
            pltpu.SemaphoreType.DMA,
        ],
        name="rtdetr_post_topk_sc",
    )(bar1d, pat1d, quads2, scale16)


def kernel(pred_bar_logits, pred_pat_logits, pred_quads, orig_target_sizes):
    scale16 = jnp.tile(orig_target_sizes, (1, 8)).reshape(-1)
    qout, blab, bsc, plab, psc = _run(
        pred_bar_logits, pred_pat_logits, pred_quads, scale16)
    quads = qout.reshape(NB, KPAD, 8)[:, :K, :]
    return (quads,
            blab.reshape(NB, KPAD)[:, :K],
            bsc.reshape(NB, KPAD)[:, :K],
            plab.reshape(NB, KPAD)[:, :K],
            psc.reshape(NB, KPAD)[:, :K])
